# Initial kernel scaffold; baseline (speedup 1.0000x reference)
import jax
import jax.numpy as jnp
from jax.experimental import pallas as pl


def kernel(xyz, features, W0, gamma0, beta0, W1, gamma1, beta1, W2, gamma2, beta2):
    raise NotImplementedError("write your pallas kernel here")


# speedup vs baseline: 1.5120x; 1.5120x over previous
import functools

import jax
import jax.numpy as jnp
from jax.experimental import pallas as pl
from jax.experimental.pallas import tpu as pltpu

N_POINT = 1024
N_SAMPLE = 32
BALL_RADIUS = 0.2
_B = 16
_N = 4096


def _fps_body(x_ref, y_ref, z_ref, f0_ref, idx_ref, cx_ref, cy_ref, cz_ref,
              dist_ref):
    X = x_ref[:, :]
    Y = y_ref[:, :]
    Z = z_ref[:, :]
    dist_ref[:, :] = jnp.full((_B, _N), 1e10, dtype=jnp.float32)
    iota = jax.lax.broadcasted_iota(jnp.int32, (_B, _N), 1)
    iota_s = jax.lax.broadcasted_iota(jnp.int32, (_B, N_POINT), 1)

    def body(i, far):
        oh = iota == far
        cx = jnp.sum(jnp.where(oh, X, 0.0), axis=1, keepdims=True)
        cy = jnp.sum(jnp.where(oh, Y, 0.0), axis=1, keepdims=True)
        cz = jnp.sum(jnp.where(oh, Z, 0.0), axis=1, keepdims=True)
        d = ((X - cx) ** 2 + (Y - cy) ** 2) + (Z - cz) ** 2
        nd = jnp.minimum(dist_ref[:, :], d)
        dist_ref[:, :] = nd
        m = jnp.max(nd, axis=1, keepdims=True)
        newfar = jnp.min(jnp.where(nd == m, iota, _N), axis=1,
                         keepdims=True).astype(jnp.int32)
        sel = iota_s == i
        idx_ref[:, :] = jnp.where(sel, far, idx_ref[:, :])
        cx_ref[:, :] = jnp.where(sel, cx, cx_ref[:, :])
        cy_ref[:, :] = jnp.where(sel, cy, cy_ref[:, :])
        cz_ref[:, :] = jnp.where(sel, cz, cz_ref[:, :])
        return newfar

    jax.lax.fori_loop(0, N_POINT, body, f0_ref[:, :1], unroll=False)


def _run_fps(xyz):
    X = xyz[:, :, 0]
    Y = xyz[:, :, 1]
    Z = xyz[:, :, 2]
    f0 = jax.random.randint(jax.random.key(42), (_B,), 0, _N).astype(jnp.int32)
    f0 = jnp.broadcast_to(f0[:, None], (_B, 128))
    out_shapes = (
        jax.ShapeDtypeStruct((_B, N_POINT), jnp.int32),
        jax.ShapeDtypeStruct((_B, N_POINT), jnp.float32),
        jax.ShapeDtypeStruct((_B, N_POINT), jnp.float32),
        jax.ShapeDtypeStruct((_B, N_POINT), jnp.float32),
    )
    idx, cx, cy, cz = pl.pallas_call(
        _fps_body,
        out_shape=out_shapes,
        scratch_shapes=[pltpu.VMEM((_B, _N), jnp.float32)],
    )(X, Y, Z, f0)
    new_xyz = jnp.stack([cx, cy, cz], axis=-1)
    return idx, new_xyz


def _index_points(points, idx):
    return jax.vmap(lambda p, i: p[i])(points, idx)


def _sq_dist(src, dst):
    d = (jnp.sum(src ** 2, axis=-1, keepdims=True)
         + jnp.swapaxes(jnp.sum(dst ** 2, axis=-1, keepdims=True), 1, 2)
         - 2.0 * jnp.matmul(src, jnp.swapaxes(dst, 1, 2)))
    return jnp.clip(d, 0.0, None)


def _ball_query(xyz, new_xyz):
    sqrdists = _sq_dist(new_xyz, xyz)
    r2 = BALL_RADIUS ** 2
    masked = jnp.where(sqrdists > r2, 1e10, sqrdists)
    _, group_idx = jax.lax.top_k(-masked, N_SAMPLE)
    gathered = jnp.take_along_axis(sqrdists, group_idx, axis=2)
    first = jnp.broadcast_to(group_idx[:, :, 0:1], group_idx.shape)
    return jnp.where(gathered > r2, first, group_idx)


def kernel(xyz, features, W0, gamma0, beta0, W1, gamma1, beta1, W2, gamma2,
           beta2):
    fps_idx, new_xyz = _run_fps(xyz)
    idx = _ball_query(xyz, new_xyz)
    grouped_xyz = _index_points(xyz, idx) - new_xyz[:, :, None, :]
    grouped_feat = _index_points(features, idx)
    grouped = jnp.concatenate([grouped_xyz, grouped_feat], axis=-1)
    g = jnp.transpose(grouped, (0, 3, 2, 1))
    for W, gm, bt in ((W0, gamma0, beta0), (W1, gamma1, beta1),
                      (W2, gamma2, beta2)):
        g = jnp.einsum('oc,bcns->bons', W, g)
        mean = jnp.mean(g, axis=(0, 2, 3), keepdims=True)
        var = jnp.var(g, axis=(0, 2, 3), keepdims=True)
        g = (g - mean) / jnp.sqrt(var + 1e-5)
        g = g * gm.reshape(1, -1, 1, 1) + bt.reshape(1, -1, 1, 1)
        g = jax.nn.relu(g)
    new_features = jnp.max(g, axis=2)
    return (new_xyz, jnp.transpose(new_features, (0, 2, 1)))


# speedup vs baseline: 3.2041x; 2.1192x over previous
import functools

import jax
import jax.numpy as jnp
from jax import lax
from jax.experimental import pallas as pl
from jax.experimental.pallas import tpu as pltpu
from jax.experimental.pallas import tpu_sc as plsc

N_POINT = 1024
N_SAMPLE = 32
BALL_RADIUS = 0.2
_B = 16
_N = 4096
_D = 80
_ROWS = _B * N_POINT * N_SAMPLE
_NW = 32
_CH = 128
_FIRE = 4
_TILE = 8192
_QT = 256
_GRID = _ROWS // _TILE


def _fps_body(x_ref, y_ref, z_ref, f0_ref, idx_ref, cx_ref, cy_ref, cz_ref,
              dist_ref):
    X = x_ref[:, :]
    Y = y_ref[:, :]
    Z = z_ref[:, :]
    dist_ref[:, :] = jnp.full((_B, _N), 1e10, dtype=jnp.float32)
    iota = jax.lax.broadcasted_iota(jnp.int32, (_B, _N), 1)
    iota_s = jax.lax.broadcasted_iota(jnp.int32, (_B, N_POINT), 1)

    def body(i, far):
        oh = iota == far
        cx = jnp.sum(jnp.where(oh, X, 0.0), axis=1, keepdims=True)
        cy = jnp.sum(jnp.where(oh, Y, 0.0), axis=1, keepdims=True)
        cz = jnp.sum(jnp.where(oh, Z, 0.0), axis=1, keepdims=True)
        d = ((X - cx) ** 2 + (Y - cy) ** 2) + (Z - cz) ** 2
        nd = jnp.minimum(dist_ref[:, :], d)
        dist_ref[:, :] = nd
        m = jnp.max(nd, axis=1, keepdims=True)
        newfar = jnp.min(jnp.where(nd == m, iota, _N), axis=1,
                         keepdims=True).astype(jnp.int32)
        sel = iota_s == i
        idx_ref[:, :] = jnp.where(sel, far, idx_ref[:, :])
        cx_ref[:, :] = jnp.where(sel, cx, cx_ref[:, :])
        cy_ref[:, :] = jnp.where(sel, cy, cy_ref[:, :])
        cz_ref[:, :] = jnp.where(sel, cz, cz_ref[:, :])
        return newfar

    jax.lax.fori_loop(0, N_POINT, body, f0_ref[:, :1], unroll=False)


def _run_fps(xyz):
    X = xyz[:, :, 0]
    Y = xyz[:, :, 1]
    Z = xyz[:, :, 2]
    f0 = jax.random.randint(jax.random.key(42), (_B,), 0, _N).astype(jnp.int32)
    f0 = jnp.broadcast_to(f0[:, None], (_B, 128))
    out_shapes = (
        jax.ShapeDtypeStruct((_B, N_POINT), jnp.int32),
        jax.ShapeDtypeStruct((_B, N_POINT), jnp.float32),
        jax.ShapeDtypeStruct((_B, N_POINT), jnp.float32),
        jax.ShapeDtypeStruct((_B, N_POINT), jnp.float32),
    )
    idx, cx, cy, cz = pl.pallas_call(
        _fps_body,
        out_shape=out_shapes,
        scratch_shapes=[pltpu.VMEM((_B, _N), jnp.float32)],
    )(X, Y, Z, f0)
    new_xyz = jnp.stack([cx, cy, cz], axis=-1)
    return idx, (cx, cy, cz), new_xyz


def _sq_dist(src, dst):
    d = (jnp.sum(src ** 2, axis=-1, keepdims=True)
         + jnp.swapaxes(jnp.sum(dst ** 2, axis=-1, keepdims=True), 1, 2)
         - 2.0 * jnp.matmul(src, jnp.swapaxes(dst, 1, 2)))
    return jnp.clip(d, 0.0, None)


def _ball_query(xyz, new_xyz):
    sqrdists = _sq_dist(new_xyz, xyz)
    r2 = BALL_RADIUS ** 2
    masked = jnp.where(sqrdists > r2, 1e10, sqrdists)
    _, group_idx = jax.lax.top_k(-masked, N_SAMPLE)
    gathered = jnp.take_along_axis(sqrdists, group_idx, axis=2)
    first = jnp.broadcast_to(group_idx[:, :, 0:1], group_idx.shape)
    return jnp.where(gathered > r2, first, group_idx)


_PER_W = _ROWS // _NW
_NSUPER = _PER_W // (_CH * _FIRE)


def _sc_gather_body(tbl_hbm, idx_hbm, out_hbm, idx_v, rows_v, gsem):
    wid = lax.axis_index("s") * 2 + lax.axis_index("c")
    nchunks = _PER_W // _CH
    pltpu.sync_copy(idx_hbm.at[pl.ds(wid * nchunks, nchunks)], idx_v)

    def super_body(s, _):
        for j in range(_FIRE):
            pltpu.async_copy(tbl_hbm.at[idx_v.at[s * _FIRE + j]],
                             rows_v.at[j], gsem)
        for j in range(_FIRE):
            pltpu.make_async_copy(tbl_hbm.at[idx_v.at[s * _FIRE + j]],
                                  rows_v.at[j], gsem).wait()
        base = wid * _PER_W + s * (_CH * _FIRE)
        for j in range(_FIRE):
            pltpu.sync_copy(rows_v.at[j],
                            out_hbm.at[pl.ds(base + j * _CH, _CH)])
        return 0

    lax.fori_loop(0, _NSUPER, super_body, 0, unroll=False)


def _sc_gather(tbl, flat_idx):
    mesh = plsc.VectorSubcoreMesh(core_axis_name="c", subcore_axis_name="s")
    fn = pl.kernel(
        _sc_gather_body,
        out_type=jax.ShapeDtypeStruct((_ROWS, _D), jnp.float32),
        mesh=mesh,
        compiler_params=pltpu.CompilerParams(use_tc_tiling_on_sc=False),
        scratch_types=[
            pltpu.VMEM((_PER_W // _CH, _CH), jnp.int32),
            pltpu.VMEM((_FIRE, _CH, _D), jnp.float32),
            pltpu.SemaphoreType.DMA,
        ],
    )
    return fn(tbl, flat_idx.reshape(_ROWS // _CH, _CH))


def _mlp1_body(g_ref, c_ref, w0f_ref, w0x_ref, y_ref, st_ref):
    g = g_ref[:, :]
    gx = g[:, :16]
    gf = g[:, 16:]
    y = (jnp.dot(gf, w0f_ref[:, :], preferred_element_type=jnp.float32)
         + jnp.dot(gx, w0x_ref[:, :], preferred_element_type=jnp.float32))
    bias = jnp.dot(c_ref[0], w0x_ref[:, :],
                   preferred_element_type=jnp.float32)
    y = (y.reshape(_QT, N_SAMPLE, 64) - bias[:, None, :]).reshape(_TILE, 64)
    y_ref[:, :] = y
    s1 = jnp.sum(y, axis=0, keepdims=True)
    s2 = jnp.sum(y * y, axis=0, keepdims=True)
    upd = jnp.concatenate([s1, s2, jnp.zeros((6, 64), jnp.float32)], axis=0)

    @pl.when(pl.program_id(0) == 0)
    def _():
        st_ref[:, :] = jnp.zeros_like(st_ref)

    st_ref[:, :] += upd


def _mlp_mid_body(y_ref, ab_ref, w_ref, o_ref, st_ref):
    a = ab_ref[0:1, :]
    b = ab_ref[1:2, :]
    h = jnp.maximum(y_ref[:, :] * a + b, 0.0)
    y = jnp.dot(h, w_ref[:, :], preferred_element_type=jnp.float32)
    o_ref[:, :] = y
    s1 = jnp.sum(y, axis=0, keepdims=True)
    s2 = jnp.sum(y * y, axis=0, keepdims=True)
    upd = jnp.concatenate([s1, s2, jnp.zeros((6, 64), jnp.float32)], axis=0)

    @pl.when(pl.program_id(0) == 0)
    def _():
        st_ref[:, :] = jnp.zeros_like(st_ref)

    st_ref[:, :] += upd


def _mlp3_body(y_ref, ab_ref, w_ref, m_ref, st_ref):
    a = ab_ref[0:1, :]
    b = ab_ref[1:2, :]
    h = jnp.maximum(y_ref[:, :] * a + b, 0.0)
    y = jnp.dot(h, w_ref[:, :], preferred_element_type=jnp.float32)
    m_ref[:, :] = jnp.max(y.reshape(_QT, N_SAMPLE, 128), axis=1)
    s1 = jnp.sum(y, axis=0, keepdims=True)
    s2 = jnp.sum(y * y, axis=0, keepdims=True)
    upd = jnp.concatenate([s1, s2, jnp.zeros((6, 128), jnp.float32)], axis=0)

    @pl.when(pl.program_id(0) == 0)
    def _():
        st_ref[:, :] = jnp.zeros_like(st_ref)

    st_ref[:, :] += upd


def _mlp4_body(m_ref, ab_ref, o_ref):
    a = ab_ref[0:1, :]
    b = ab_ref[1:2, :]
    o_ref[:, :] = jnp.maximum(m_ref[:, :] * a + b, 0.0)


def _bn_ab(st, gamma, beta):
    cnt = jnp.float32(_ROWS)
    mean = st[0] / cnt
    var = st[1] / cnt - mean * mean
    a = gamma / jnp.sqrt(var + 1e-5)
    b = beta - mean * a
    return a, b


def _pack_ab(a, b, width):
    ab = jnp.stack([a, b], axis=0)
    return jnp.concatenate([ab, jnp.zeros((6, width), jnp.float32)], axis=0)


def _run_mlp(grouped, new_xyz_pad, W0, gamma0, beta0, W1, gamma1, beta1,
             W2, gamma2, beta2):
    w0 = W0.T
    w0x = jnp.concatenate([w0[:3], jnp.zeros((13, 64), jnp.float32)], axis=0)
    w0f = w0[3:]
    y1, st1 = pl.pallas_call(
        _mlp1_body,
        grid=(_GRID,),
        in_specs=[
            pl.BlockSpec((_TILE, _D), lambda g: (g, 0)),
            pl.BlockSpec((1, _QT, 16), lambda g: (g // 4, g % 4, 0)),
            pl.BlockSpec((64, 64), lambda g: (0, 0)),
            pl.BlockSpec((16, 64), lambda g: (0, 0)),
        ],
        out_specs=(
            pl.BlockSpec((_TILE, 64), lambda g: (g, 0)),
            pl.BlockSpec((8, 64), lambda g: (0, 0)),
        ),
        out_shape=(
            jax.ShapeDtypeStruct((_ROWS, 64), jnp.float32),
            jax.ShapeDtypeStruct((8, 64), jnp.float32),
        ),
    )(grouped, new_xyz_pad, w0f, w0x)
    a1, b1 = _bn_ab(st1, gamma0, beta0)

    y2, st2 = pl.pallas_call(
        _mlp_mid_body,
        grid=(_GRID,),
        in_specs=[
            pl.BlockSpec((_TILE, 64), lambda g: (g, 0)),
            pl.BlockSpec((8, 64), lambda g: (0, 0)),
            pl.BlockSpec((64, 64), lambda g: (0, 0)),
        ],
        out_specs=(
            pl.BlockSpec((_TILE, 64), lambda g: (g, 0)),
            pl.BlockSpec((8, 64), lambda g: (0, 0)),
        ),
        out_shape=(
            jax.ShapeDtypeStruct((_ROWS, 64), jnp.float32),
            jax.ShapeDtypeStruct((8, 64), jnp.float32),
        ),
    )(y1, _pack_ab(a1, b1, 64), W1.T)
    a2, b2 = _bn_ab(st2, gamma1, beta1)

    m, st3 = pl.pallas_call(
        _mlp3_body,
        grid=(_GRID,),
        in_specs=[
            pl.BlockSpec((_TILE, 64), lambda g: (g, 0)),
            pl.BlockSpec((8, 64), lambda g: (0, 0)),
            pl.BlockSpec((64, 128), lambda g: (0, 0)),
        ],
        out_specs=(
            pl.BlockSpec((_QT, 128), lambda g: (g, 0)),
            pl.BlockSpec((8, 128), lambda g: (0, 0)),
        ),
        out_shape=(
            jax.ShapeDtypeStruct((_B * N_POINT, 128), jnp.float32),
            jax.ShapeDtypeStruct((8, 128), jnp.float32),
        ),
    )(y2, _pack_ab(a2, b2, 64), W2.T)
    a3, b3 = _bn_ab(st3, gamma2, beta2)

    out = pl.pallas_call(
        _mlp4_body,
        grid=(_GRID,),
        in_specs=[
            pl.BlockSpec((_QT, 128), lambda g: (g, 0)),
            pl.BlockSpec((8, 128), lambda g: (0, 0)),
        ],
        out_specs=pl.BlockSpec((_QT, 128), lambda g: (g, 0)),
        out_shape=jax.ShapeDtypeStruct((_B * N_POINT, 128), jnp.float32),
    )(m, _pack_ab(a3, b3, 128))
    return out.reshape(_B, N_POINT, 128)


def kernel(xyz, features, W0, gamma0, beta0, W1, gamma1, beta1, W2, gamma2,
           beta2):
    fps_idx, (cx, cy, cz), new_xyz = _run_fps(xyz)
    idx = _ball_query(xyz, new_xyz)

    tbl = jnp.concatenate(
        [xyz, jnp.zeros((_B, _N, 13), jnp.float32), features],
        axis=-1).reshape(_B * _N, _D)
    offs = (jnp.arange(_B, dtype=jnp.int32) * _N)[:, None, None]
    flat_idx = (idx + offs).reshape(_ROWS)
    grouped = _sc_gather(tbl, flat_idx)

    new_xyz_pad = jnp.concatenate(
        [new_xyz, jnp.zeros((_B, N_POINT, 13), jnp.float32)], axis=-1)
    new_features = _run_mlp(grouped, new_xyz_pad, W0, gamma0, beta0,
                            W1, gamma1, beta1, W2, gamma2, beta2)
    return (new_xyz, new_features)


# speedup vs baseline: 13.7466x; 4.2904x over previous
import functools

import jax
import jax.numpy as jnp
from jax import lax
from jax.experimental import pallas as pl
from jax.experimental.pallas import tpu as pltpu
from jax.experimental.pallas import tpu_sc as plsc

N_POINT = 1024
N_SAMPLE = 32
BALL_RADIUS = 0.2
_B = 16
_N = 4096
_D = 80
_ROWS = _B * N_POINT * N_SAMPLE
_NW = 32
_CH = 128
_FIRE = 4
_TILE = 8192
_QT = 256
_GRID = _ROWS // _TILE


def _fps_body(x_ref, y_ref, z_ref, f0_ref, idx_ref, cx_ref, cy_ref, cz_ref,
              dist_ref):
    X = x_ref[:, :]
    Y = y_ref[:, :]
    Z = z_ref[:, :]
    dist_ref[:, :] = jnp.full((_B, _N), 1e10, dtype=jnp.float32)
    iota = jax.lax.broadcasted_iota(jnp.int32, (_B, _N), 1)
    iota_s = jax.lax.broadcasted_iota(jnp.int32, (_B, N_POINT), 1)

    def body(i, far):
        oh = iota == far
        cx = jnp.sum(jnp.where(oh, X, 0.0), axis=1, keepdims=True)
        cy = jnp.sum(jnp.where(oh, Y, 0.0), axis=1, keepdims=True)
        cz = jnp.sum(jnp.where(oh, Z, 0.0), axis=1, keepdims=True)
        d = ((X - cx) ** 2 + (Y - cy) ** 2) + (Z - cz) ** 2
        nd = jnp.minimum(dist_ref[:, :], d)
        dist_ref[:, :] = nd
        m = jnp.max(nd, axis=1, keepdims=True)
        newfar = jnp.min(jnp.where(nd == m, iota, _N), axis=1,
                         keepdims=True).astype(jnp.int32)
        sel = iota_s == i
        idx_ref[:, :] = jnp.where(sel, far, idx_ref[:, :])
        cx_ref[:, :] = jnp.where(sel, cx, cx_ref[:, :])
        cy_ref[:, :] = jnp.where(sel, cy, cy_ref[:, :])
        cz_ref[:, :] = jnp.where(sel, cz, cz_ref[:, :])
        return newfar

    jax.lax.fori_loop(0, N_POINT, body, f0_ref[:, :1], unroll=False)


def _run_fps(xyz):
    X = xyz[:, :, 0]
    Y = xyz[:, :, 1]
    Z = xyz[:, :, 2]
    f0 = jax.random.randint(jax.random.key(42), (_B,), 0, _N).astype(jnp.int32)
    f0 = jnp.broadcast_to(f0[:, None], (_B, 128))
    out_shapes = (
        jax.ShapeDtypeStruct((_B, N_POINT), jnp.int32),
        jax.ShapeDtypeStruct((_B, N_POINT), jnp.float32),
        jax.ShapeDtypeStruct((_B, N_POINT), jnp.float32),
        jax.ShapeDtypeStruct((_B, N_POINT), jnp.float32),
    )
    idx, cx, cy, cz = pl.pallas_call(
        _fps_body,
        out_shape=out_shapes,
        scratch_shapes=[pltpu.VMEM((_B, _N), jnp.float32)],
    )(X, Y, Z, f0)
    new_xyz = jnp.stack([cx, cy, cz], axis=-1)
    return idx, (cx, cy, cz), new_xyz


_R2 = BALL_RADIUS * BALL_RADIUS
_QTOT = _B * N_POINT
_QPW = _QTOT // _NW
_NCHUNK = 64
_CPQ = _N // _NCHUNK


def _dist_body(q8_ref, p8_ref, md_ref):
    q8 = q8_ref[0]
    p8 = p8_ref[0]
    colmask = lax.broadcasted_iota(jnp.int32, (_QT, 8), 1) < 3
    rowmask = lax.broadcasted_iota(jnp.int32, (8, _N), 0) < 3
    qb = jnp.where(colmask, q8, 0.0).astype(jnp.bfloat16)
    pb = jnp.where(rowmask, p8, 0.0).astype(jnp.bfloat16)
    mm2 = jnp.dot(qb, pb, preferred_element_type=jnp.float32)
    qq = q8[:, 3:4]
    pp = p8[4:5, :]
    d = (qq + pp) + mm2
    d = jnp.maximum(d, 0.0)
    md_ref[:, :] = jnp.where(d > _R2, 1e10, d)


def _masked_dists(q8, p8):
    return pl.pallas_call(
        _dist_body,
        grid=(_GRID,),
        in_specs=[
            pl.BlockSpec((1, _QT, 8), lambda g: (g // 4, g % 4, 0)),
            pl.BlockSpec((1, 8, _N), lambda g: (g // 4, 0, 0)),
        ],
        out_specs=pl.BlockSpec((_QT, _N), lambda g: (g, 0)),
        out_shape=jax.ShapeDtypeStruct((_QTOT, _N), jnp.float32),
    )(q8, p8)


def _select_process(d_at, oi_ref, ql):
    lane = lax.broadcasted_iota(jnp.int32, (16,), 0)
    inf16 = jnp.full((16,), 1e10, jnp.float32)
    cbase = [(lane + 16 * jv) * _NCHUNK for jv in range(4)]

    def cmloop(s, cms):
        return tuple(
            jnp.minimum(cms[jv], plsc.load_gather(d_at, [cbase[jv] + s]))
            for jv in range(4))

    cms0 = lax.fori_loop(0, _NCHUNK, cmloop, (inf16,) * 4, unroll=8)

    def exloop(j, carry):
        i0, cm0, cm1, cm2, cm3 = carry[:5]
        o0, o1 = carry[5:]
        cms = [cm0, cm1, cm2, cm3]
        best = inf16
        brow = jnp.zeros((16,), jnp.int32)
        for jv in range(4):
            upd = cms[jv] < best
            best = jnp.where(upd, cms[jv], best)
            brow = jnp.where(upd, jnp.full((16,), jv, jnp.int32), brow)
        gm = jnp.min(best)
        chunkid = brow * 16 + lane
        cstar = jnp.min(jnp.where(best == gm, chunkid,
                                  jnp.full((16,), 10 ** 6, jnp.int32)))
        big = jnp.full((16,), 10 ** 9, jnp.int32)
        posmin = big
        col = []
        base = cstar * _NCHUNK
        for k in range(4):
            pos = base + 16 * k + lane
            wk = d_at[pl.ds(base + 16 * k, 16)]
            col.append((pos, wk))
            posmin = jnp.minimum(posmin, jnp.where(wk == gm, pos, big))
        gidx_raw = jnp.min(posmin)
        ncm16 = inf16
        for k in range(4):
            pos, wk = col[k]
            ncm16 = jnp.minimum(ncm16, jnp.where(pos == gidx_raw, inf16, wk))
        plsc.store_scatter(d_at, [jnp.full((16,), gidx_raw, jnp.int32)],
                           inf16, mask=lane == 0)
        ncm = jnp.min(ncm16)
        ncms = []
        for jv in range(4):
            chunk_sel = (lane + 16 * jv) == cstar
            ncms.append(jnp.where(chunk_sel,
                                  jnp.full((16,), ncm, jnp.float32), cms[jv]))
        i0_new = jnp.where(j == 0, gidx_raw, i0)
        gidx = jnp.where(gm > _R2, i0_new, gidx_raw)
        o0 = jnp.where(lane == j, jnp.full((16,), gidx, jnp.int32), o0)
        o1 = jnp.where(lane == (j - 16), jnp.full((16,), gidx, jnp.int32), o1)
        return (i0_new, ncms[0], ncms[1], ncms[2], ncms[3], o0, o1)

    zero16 = jnp.zeros((16,), jnp.int32)
    res = lax.fori_loop(
        0, N_SAMPLE, exloop,
        (jnp.int32(0),) + cms0 + (zero16, zero16), unroll=False)
    oi_ref[pl.ds(ql * N_SAMPLE, 16)] = res[5]
    oi_ref[pl.ds(ql * N_SAMPLE + 16, 16)] = res[6]


def _select_body(md_hbm, oidx_hbm, d_v, oi_v, sem0, sem1):
    w = lax.axis_index("s") * 2 + lax.axis_index("c")
    q0 = w * _QPW
    pltpu.async_copy(md_hbm.at[q0], d_v.at[0], sem0)

    def qloop(t, _):
        q = q0 + 2 * t
        pltpu.make_async_copy(md_hbm.at[q], d_v.at[0], sem0).wait()
        pltpu.async_copy(md_hbm.at[q + 1], d_v.at[1], sem1)
        _select_process(d_v.at[0], oi_v, 2 * t)

        @pl.when(t < _QPW // 2 - 1)
        def _():
            pltpu.async_copy(md_hbm.at[q + 2], d_v.at[0], sem0)

        pltpu.make_async_copy(md_hbm.at[q + 1], d_v.at[1], sem1).wait()
        _select_process(d_v.at[1], oi_v, 2 * t + 1)
        return 0

    lax.fori_loop(0, _QPW // 2, qloop, 0, unroll=False)
    pltpu.sync_copy(oi_v, oidx_hbm.at[pl.ds(q0 * N_SAMPLE,
                                            _QPW * N_SAMPLE)])


def _sc_select(md):
    mesh = plsc.VectorSubcoreMesh(core_axis_name="c", subcore_axis_name="s")
    fn = pl.kernel(
        _select_body,
        out_type=jax.ShapeDtypeStruct((_QTOT * N_SAMPLE,), jnp.int32),
        mesh=mesh,
        compiler_params=pltpu.CompilerParams(use_tc_tiling_on_sc=False,
                                             needs_layout_passes=False),
        scratch_types=[
            pltpu.VMEM((2, _N), jnp.float32),
            pltpu.VMEM((_QPW * N_SAMPLE,), jnp.int32),
            pltpu.SemaphoreType.DMA,
            pltpu.SemaphoreType.DMA,
        ],
    )
    return fn(md)


def _ball_query(xyz, new_xyz, cxyz):
    cx, cy, cz = cxyz
    X = xyz[:, :, 0]
    Y = xyz[:, :, 1]
    Z = xyz[:, :, 2]
    pp = X * X + Y * Y + Z * Z
    qq = cx * cx + cy * cy + cz * cz
    ones_q = jnp.ones_like(cx)
    q8 = jnp.stack([cx, cy, cz, qq, ones_q,
                    jnp.zeros_like(cx), jnp.zeros_like(cx),
                    jnp.zeros_like(cx)], axis=-1)
    p8 = jnp.stack([-2.0 * X, -2.0 * Y, -2.0 * Z, jnp.ones_like(X), pp,
                    jnp.zeros_like(X), jnp.zeros_like(X),
                    jnp.zeros_like(X)], axis=1)
    md = _masked_dists(q8, p8)
    idx = _sc_select(md)
    return idx.reshape(_B, N_POINT, N_SAMPLE)


_PER_W = _ROWS // _NW
_NSUPER = _PER_W // (_CH * _FIRE)


def _sc_gather_body(tbl_hbm, idx_hbm, out_hbm, idx_v, rows_v, gsem):
    wid = lax.axis_index("s") * 2 + lax.axis_index("c")
    nchunks = _PER_W // _CH
    pltpu.sync_copy(idx_hbm.at[pl.ds(wid * nchunks, nchunks)], idx_v)

    def super_body(s, _):
        for j in range(_FIRE):
            pltpu.async_copy(tbl_hbm.at[idx_v.at[s * _FIRE + j]],
                             rows_v.at[j], gsem)
        for j in range(_FIRE):
            pltpu.make_async_copy(tbl_hbm.at[idx_v.at[s * _FIRE + j]],
                                  rows_v.at[j], gsem).wait()
        base = wid * _PER_W + s * (_CH * _FIRE)
        for j in range(_FIRE):
            pltpu.sync_copy(rows_v.at[j],
                            out_hbm.at[pl.ds(base + j * _CH, _CH)])
        return 0

    lax.fori_loop(0, _NSUPER, super_body, 0, unroll=False)


def _sc_gather(tbl, flat_idx):
    mesh = plsc.VectorSubcoreMesh(core_axis_name="c", subcore_axis_name="s")
    fn = pl.kernel(
        _sc_gather_body,
        out_type=jax.ShapeDtypeStruct((_ROWS, _D), jnp.float32),
        mesh=mesh,
        compiler_params=pltpu.CompilerParams(use_tc_tiling_on_sc=False),
        scratch_types=[
            pltpu.VMEM((_PER_W // _CH, _CH), jnp.int32),
            pltpu.VMEM((_FIRE, _CH, _D), jnp.float32),
            pltpu.SemaphoreType.DMA,
        ],
    )
    return fn(tbl, flat_idx.reshape(_ROWS // _CH, _CH))


def _mlp1_body(g_ref, c_ref, w0f_ref, w0x_ref, y_ref, st_ref):
    g = g_ref[:, :]
    gx = g[:, :16]
    gf = g[:, 16:]
    y = (jnp.dot(gf, w0f_ref[:, :], preferred_element_type=jnp.float32)
         + jnp.dot(gx, w0x_ref[:, :], preferred_element_type=jnp.float32))
    bias = jnp.dot(c_ref[0], w0x_ref[:, :],
                   preferred_element_type=jnp.float32)
    y = (y.reshape(_QT, N_SAMPLE, 64) - bias[:, None, :]).reshape(_TILE, 64)
    y_ref[:, :] = y
    s1 = jnp.sum(y, axis=0, keepdims=True)
    s2 = jnp.sum(y * y, axis=0, keepdims=True)
    upd = jnp.concatenate([s1, s2, jnp.zeros((6, 64), jnp.float32)], axis=0)

    @pl.when(pl.program_id(0) == 0)
    def _():
        st_ref[:, :] = jnp.zeros_like(st_ref)

    st_ref[:, :] += upd


def _mlp_mid_body(y_ref, ab_ref, w_ref, o_ref, st_ref):
    a = ab_ref[0:1, :]
    b = ab_ref[1:2, :]
    h = jnp.maximum(y_ref[:, :] * a + b, 0.0)
    y = jnp.dot(h, w_ref[:, :], preferred_element_type=jnp.float32)
    o_ref[:, :] = y
    s1 = jnp.sum(y, axis=0, keepdims=True)
    s2 = jnp.sum(y * y, axis=0, keepdims=True)
    upd = jnp.concatenate([s1, s2, jnp.zeros((6, 64), jnp.float32)], axis=0)

    @pl.when(pl.program_id(0) == 0)
    def _():
        st_ref[:, :] = jnp.zeros_like(st_ref)

    st_ref[:, :] += upd


def _mlp3_body(y_ref, ab_ref, w_ref, m_ref, st_ref):
    a = ab_ref[0:1, :]
    b = ab_ref[1:2, :]
    h = jnp.maximum(y_ref[:, :] * a + b, 0.0)
    y = jnp.dot(h, w_ref[:, :], preferred_element_type=jnp.float32)
    m_ref[:, :] = jnp.max(y.reshape(_QT, N_SAMPLE, 128), axis=1)
    s1 = jnp.sum(y, axis=0, keepdims=True)
    s2 = jnp.sum(y * y, axis=0, keepdims=True)
    upd = jnp.concatenate([s1, s2, jnp.zeros((6, 128), jnp.float32)], axis=0)

    @pl.when(pl.program_id(0) == 0)
    def _():
        st_ref[:, :] = jnp.zeros_like(st_ref)

    st_ref[:, :] += upd


def _mlp4_body(m_ref, ab_ref, o_ref):
    a = ab_ref[0:1, :]
    b = ab_ref[1:2, :]
    o_ref[:, :] = jnp.maximum(m_ref[:, :] * a + b, 0.0)


def _bn_ab(st, gamma, beta):
    cnt = jnp.float32(_ROWS)
    mean = st[0] / cnt
    var = st[1] / cnt - mean * mean
    a = gamma / jnp.sqrt(var + 1e-5)
    b = beta - mean * a
    return a, b


def _pack_ab(a, b, width):
    ab = jnp.stack([a, b], axis=0)
    return jnp.concatenate([ab, jnp.zeros((6, width), jnp.float32)], axis=0)


def _run_mlp(grouped, new_xyz_pad, W0, gamma0, beta0, W1, gamma1, beta1,
             W2, gamma2, beta2):
    w0 = W0.T
    w0x = jnp.concatenate([w0[:3], jnp.zeros((13, 64), jnp.float32)], axis=0)
    w0f = w0[3:]
    y1, st1 = pl.pallas_call(
        _mlp1_body,
        grid=(_GRID,),
        in_specs=[
            pl.BlockSpec((_TILE, _D), lambda g: (g, 0)),
            pl.BlockSpec((1, _QT, 16), lambda g: (g // 4, g % 4, 0)),
            pl.BlockSpec((64, 64), lambda g: (0, 0)),
            pl.BlockSpec((16, 64), lambda g: (0, 0)),
        ],
        out_specs=(
            pl.BlockSpec((_TILE, 64), lambda g: (g, 0)),
            pl.BlockSpec((8, 64), lambda g: (0, 0)),
        ),
        out_shape=(
            jax.ShapeDtypeStruct((_ROWS, 64), jnp.float32),
            jax.ShapeDtypeStruct((8, 64), jnp.float32),
        ),
    )(grouped, new_xyz_pad, w0f, w0x)
    a1, b1 = _bn_ab(st1, gamma0, beta0)

    y2, st2 = pl.pallas_call(
        _mlp_mid_body,
        grid=(_GRID,),
        in_specs=[
            pl.BlockSpec((_TILE, 64), lambda g: (g, 0)),
            pl.BlockSpec((8, 64), lambda g: (0, 0)),
            pl.BlockSpec((64, 64), lambda g: (0, 0)),
        ],
        out_specs=(
            pl.BlockSpec((_TILE, 64), lambda g: (g, 0)),
            pl.BlockSpec((8, 64), lambda g: (0, 0)),
        ),
        out_shape=(
            jax.ShapeDtypeStruct((_ROWS, 64), jnp.float32),
            jax.ShapeDtypeStruct((8, 64), jnp.float32),
        ),
    )(y1, _pack_ab(a1, b1, 64), W1.T)
    a2, b2 = _bn_ab(st2, gamma1, beta1)

    m, st3 = pl.pallas_call(
        _mlp3_body,
        grid=(_GRID,),
        in_specs=[
            pl.BlockSpec((_TILE, 64), lambda g: (g, 0)),
            pl.BlockSpec((8, 64), lambda g: (0, 0)),
            pl.BlockSpec((64, 128), lambda g: (0, 0)),
        ],
        out_specs=(
            pl.BlockSpec((_QT, 128), lambda g: (g, 0)),
            pl.BlockSpec((8, 128), lambda g: (0, 0)),
        ),
        out_shape=(
            jax.ShapeDtypeStruct((_B * N_POINT, 128), jnp.float32),
            jax.ShapeDtypeStruct((8, 128), jnp.float32),
        ),
    )(y2, _pack_ab(a2, b2, 64), W2.T)
    a3, b3 = _bn_ab(st3, gamma2, beta2)

    out = pl.pallas_call(
        _mlp4_body,
        grid=(_GRID,),
        in_specs=[
            pl.BlockSpec((_QT, 128), lambda g: (g, 0)),
            pl.BlockSpec((8, 128), lambda g: (0, 0)),
        ],
        out_specs=pl.BlockSpec((_QT, 128), lambda g: (g, 0)),
        out_shape=jax.ShapeDtypeStruct((_B * N_POINT, 128), jnp.float32),
    )(m, _pack_ab(a3, b3, 128))
    return out.reshape(_B, N_POINT, 128)


def kernel(xyz, features, W0, gamma0, beta0, W1, gamma1, beta1, W2, gamma2,
           beta2):
    fps_idx, (cx, cy, cz), new_xyz = _run_fps(xyz)
    idx = _ball_query(xyz, new_xyz, (cx, cy, cz))

    tbl = jnp.concatenate(
        [xyz, jnp.zeros((_B, _N, 13), jnp.float32), features],
        axis=-1).reshape(_B * _N, _D)
    offs = (jnp.arange(_B, dtype=jnp.int32) * _N)[:, None, None]
    flat_idx = (idx + offs).reshape(_ROWS)
    grouped = _sc_gather(tbl, flat_idx)

    new_xyz_pad = jnp.concatenate(
        [new_xyz, jnp.zeros((_B, N_POINT, 13), jnp.float32)], axis=-1)
    new_features = _run_mlp(grouped, new_xyz_pad, W0, gamma0, beta0,
                            W1, gamma1, beta1, W2, gamma2, beta2)
    return (new_xyz, new_features)


# speedup vs baseline: 15.2650x; 1.1105x over previous
import functools

import jax
import jax.numpy as jnp
from jax import lax
from jax.experimental import pallas as pl
from jax.experimental.pallas import tpu as pltpu
from jax.experimental.pallas import tpu_sc as plsc

N_POINT = 1024
N_SAMPLE = 32
BALL_RADIUS = 0.2
_B = 16
_N = 4096
_D = 128
_ROWS = _B * N_POINT * N_SAMPLE
_NW = 32
_CH = 128
_FIRE = 4
_TILE = 8192
_QT = 256
_GRID = _ROWS // _TILE


def _fps_body(x_ref, y_ref, z_ref, f0_ref, idx_ref, cx_ref, cy_ref, cz_ref,
              dist_ref):
    X = x_ref[:, :]
    Y = y_ref[:, :]
    Z = z_ref[:, :]
    dist_ref[:, :] = jnp.full((_B, _N), 1e10, dtype=jnp.float32)
    iota = jax.lax.broadcasted_iota(jnp.int32, (_B, _N), 1)
    iota_s = jax.lax.broadcasted_iota(jnp.int32, (_B, N_POINT), 1)

    def body(i, far):
        oh = iota == far
        cx = jnp.sum(jnp.where(oh, X, 0.0), axis=1, keepdims=True)
        cy = jnp.sum(jnp.where(oh, Y, 0.0), axis=1, keepdims=True)
        cz = jnp.sum(jnp.where(oh, Z, 0.0), axis=1, keepdims=True)
        d = ((X - cx) ** 2 + (Y - cy) ** 2) + (Z - cz) ** 2
        nd = jnp.minimum(dist_ref[:, :], d)
        dist_ref[:, :] = nd
        m = jnp.max(nd, axis=1, keepdims=True)
        newfar = jnp.min(jnp.where(nd == m, iota, _N), axis=1,
                         keepdims=True).astype(jnp.int32)
        sel = iota_s == i
        idx_ref[:, :] = jnp.where(sel, far, idx_ref[:, :])
        cx_ref[:, :] = jnp.where(sel, cx, cx_ref[:, :])
        cy_ref[:, :] = jnp.where(sel, cy, cy_ref[:, :])
        cz_ref[:, :] = jnp.where(sel, cz, cz_ref[:, :])
        return newfar

    jax.lax.fori_loop(0, N_POINT, body, f0_ref[:, :1], unroll=False)


def _run_fps(xyz):
    X = xyz[:, :, 0]
    Y = xyz[:, :, 1]
    Z = xyz[:, :, 2]
    f0 = jax.random.randint(jax.random.key(42), (_B,), 0, _N).astype(jnp.int32)
    f0 = jnp.broadcast_to(f0[:, None], (_B, 128))
    out_shapes = (
        jax.ShapeDtypeStruct((_B, N_POINT), jnp.int32),
        jax.ShapeDtypeStruct((_B, N_POINT), jnp.float32),
        jax.ShapeDtypeStruct((_B, N_POINT), jnp.float32),
        jax.ShapeDtypeStruct((_B, N_POINT), jnp.float32),
    )
    idx, cx, cy, cz = pl.pallas_call(
        _fps_body,
        out_shape=out_shapes,
        scratch_shapes=[pltpu.VMEM((_B, _N), jnp.float32)],
    )(X, Y, Z, f0)
    new_xyz = jnp.stack([cx, cy, cz], axis=-1)
    return idx, (cx, cy, cz), new_xyz


_R2 = BALL_RADIUS * BALL_RADIUS
_QTOT = _B * N_POINT
_QPW = _QTOT // _NW
_NCHUNK = 64
_CPQ = _N // _NCHUNK


def _dist_body(q8_ref, p8_ref, md_ref):
    q8 = q8_ref[0]
    p8 = p8_ref[0]
    colmask = lax.broadcasted_iota(jnp.int32, (_QT, 8), 1) < 3
    rowmask = lax.broadcasted_iota(jnp.int32, (8, _N), 0) < 3
    qb = jnp.where(colmask, q8, 0.0).astype(jnp.bfloat16)
    pb = jnp.where(rowmask, p8, 0.0).astype(jnp.bfloat16)
    mm2 = jnp.dot(qb, pb, preferred_element_type=jnp.float32)
    qq = q8[:, 3:4]
    pp = p8[4:5, :]
    d = (qq + pp) + mm2
    d = jnp.maximum(d, 0.0)
    md_ref[:, :] = jnp.where(d > _R2, 1e10, d)


def _masked_dists(q8, p8):
    return pl.pallas_call(
        _dist_body,
        grid=(_GRID,),
        in_specs=[
            pl.BlockSpec((1, _QT, 8), lambda g: (g // 4, g % 4, 0)),
            pl.BlockSpec((1, 8, _N), lambda g: (g // 4, 0, 0)),
        ],
        out_specs=pl.BlockSpec((_QT, _N), lambda g: (g, 0)),
        out_shape=jax.ShapeDtypeStruct((_QTOT, _N), jnp.float32),
    )(q8, p8)


def _select_process(d_at, oi_ref, ql):
    lane = lax.broadcasted_iota(jnp.int32, (16,), 0)
    inf16 = jnp.full((16,), 1e10, jnp.float32)
    cbase = [(lane + 16 * jv) * _NCHUNK for jv in range(4)]

    def cmloop(s, cms):
        return tuple(
            jnp.minimum(cms[jv], plsc.load_gather(d_at, [cbase[jv] + s]))
            for jv in range(4))

    cms0 = lax.fori_loop(0, _NCHUNK, cmloop, (inf16,) * 4, unroll=8)

    def exloop(j, carry):
        i0, cm0, cm1, cm2, cm3 = carry[:5]
        o0, o1 = carry[5:]
        cms = [cm0, cm1, cm2, cm3]
        best = inf16
        brow = jnp.zeros((16,), jnp.int32)
        for jv in range(4):
            upd = cms[jv] < best
            best = jnp.where(upd, cms[jv], best)
            brow = jnp.where(upd, jnp.full((16,), jv, jnp.int32), brow)
        gm = jnp.min(best)
        chunkid = brow * 16 + lane
        cstar = jnp.min(jnp.where(best == gm, chunkid,
                                  jnp.full((16,), 10 ** 6, jnp.int32)))
        big = jnp.full((16,), 10 ** 9, jnp.int32)
        posmin = big
        col = []
        base = cstar * _NCHUNK
        for k in range(4):
            pos = base + 16 * k + lane
            wk = d_at[pl.ds(base + 16 * k, 16)]
            col.append((pos, wk))
            posmin = jnp.minimum(posmin, jnp.where(wk == gm, pos, big))
        gidx_raw = jnp.min(posmin)
        ncm16 = inf16
        for k in range(4):
            pos, wk = col[k]
            ncm16 = jnp.minimum(ncm16, jnp.where(pos == gidx_raw, inf16, wk))
        plsc.store_scatter(d_at, [jnp.full((16,), gidx_raw, jnp.int32)],
                           inf16, mask=lane == 0)
        ncm = jnp.min(ncm16)
        ncms = []
        for jv in range(4):
            chunk_sel = (lane + 16 * jv) == cstar
            ncms.append(jnp.where(chunk_sel,
                                  jnp.full((16,), ncm, jnp.float32), cms[jv]))
        i0_new = jnp.where(j == 0, gidx_raw, i0)
        gidx = jnp.where(gm > _R2, i0_new, gidx_raw)
        o0 = jnp.where(lane == j, jnp.full((16,), gidx, jnp.int32), o0)
        o1 = jnp.where(lane == (j - 16), jnp.full((16,), gidx, jnp.int32), o1)
        return (i0_new, ncms[0], ncms[1], ncms[2], ncms[3], o0, o1)

    zero16 = jnp.zeros((16,), jnp.int32)
    res = lax.fori_loop(
        0, N_SAMPLE, exloop,
        (jnp.int32(0),) + cms0 + (zero16, zero16), unroll=False)
    oi_ref[pl.ds(ql * N_SAMPLE, 16)] = res[5]
    oi_ref[pl.ds(ql * N_SAMPLE + 16, 16)] = res[6]


def _select_body(md_hbm, oidx_hbm, d0_v, d1_v, oi_v, sem0, sem1):
    w = lax.axis_index("s") * 2 + lax.axis_index("c")
    q0 = w * _QPW
    pltpu.async_copy(md_hbm.at[q0], d0_v, sem0)

    def qloop(t, _):
        q = q0 + 2 * t
        pltpu.make_async_copy(md_hbm.at[q], d0_v, sem0).wait()
        pltpu.async_copy(md_hbm.at[q + 1], d1_v, sem1)
        _select_process(d0_v, oi_v, 2 * t)

        @pl.when(t < _QPW // 2 - 1)
        def _():
            pltpu.async_copy(md_hbm.at[q + 2], d0_v, sem0)

        pltpu.make_async_copy(md_hbm.at[q + 1], d1_v, sem1).wait()
        _select_process(d1_v, oi_v, 2 * t + 1)
        return 0

    lax.fori_loop(0, _QPW // 2, qloop, 0, unroll=False)
    pltpu.sync_copy(oi_v, oidx_hbm.at[pl.ds(q0 * N_SAMPLE,
                                            _QPW * N_SAMPLE)])


def _sc_select(md):
    mesh = plsc.VectorSubcoreMesh(core_axis_name="c", subcore_axis_name="s")
    fn = pl.kernel(
        _select_body,
        out_type=jax.ShapeDtypeStruct((_QTOT * N_SAMPLE,), jnp.int32),
        mesh=mesh,
        compiler_params=pltpu.CompilerParams(needs_layout_passes=False),
        scratch_types=[
            pltpu.VMEM((_N,), jnp.float32),
            pltpu.VMEM((_N,), jnp.float32),
            pltpu.VMEM((_QPW * N_SAMPLE,), jnp.int32),
            pltpu.SemaphoreType.DMA,
            pltpu.SemaphoreType.DMA,
        ],
    )
    return fn(md)


def _ball_query(xyz, new_xyz, cxyz):
    cx, cy, cz = cxyz
    X = xyz[:, :, 0]
    Y = xyz[:, :, 1]
    Z = xyz[:, :, 2]
    pp = X * X + Y * Y + Z * Z
    qq = cx * cx + cy * cy + cz * cz
    ones_q = jnp.ones_like(cx)
    q8 = jnp.stack([cx, cy, cz, qq, ones_q,
                    jnp.zeros_like(cx), jnp.zeros_like(cx),
                    jnp.zeros_like(cx)], axis=-1)
    p8 = jnp.stack([-2.0 * X, -2.0 * Y, -2.0 * Z, jnp.ones_like(X), pp,
                    jnp.zeros_like(X), jnp.zeros_like(X),
                    jnp.zeros_like(X)], axis=1)
    md = _masked_dists(q8, p8)
    idx = _sc_select(md)
    return idx.reshape(_B, N_POINT, N_SAMPLE)


_PER_W = _ROWS // _NW
_NSUPER = _PER_W // (_CH * _FIRE)


def _sc_gather_body(tbl_hbm, idx_hbm, out_hbm, idx_v, rows_v, gsem):
    wid = lax.axis_index("s") * 2 + lax.axis_index("c")
    nchunks = _PER_W // _CH
    pltpu.sync_copy(idx_hbm.at[pl.ds(wid * nchunks, nchunks)], idx_v)

    def super_body(s, _):
        for j in range(_FIRE):
            pltpu.async_copy(tbl_hbm.at[idx_v.at[s * _FIRE + j]],
                             rows_v.at[j], gsem)
        for j in range(_FIRE):
            pltpu.make_async_copy(tbl_hbm.at[idx_v.at[s * _FIRE + j]],
                                  rows_v.at[j], gsem).wait()
        base = wid * _PER_W + s * (_CH * _FIRE)
        for j in range(_FIRE):
            pltpu.sync_copy(rows_v.at[j],
                            out_hbm.at[pl.ds(base + j * _CH, _CH)])
        return 0

    lax.fori_loop(0, _NSUPER, super_body, 0, unroll=False)


def _sc_gather(tbl, flat_idx):
    mesh = plsc.VectorSubcoreMesh(core_axis_name="c", subcore_axis_name="s")
    fn = pl.kernel(
        _sc_gather_body,
        out_type=jax.ShapeDtypeStruct((_ROWS, _D), jnp.float32),
        mesh=mesh,
        scratch_types=[
            pltpu.VMEM((_PER_W // _CH, _CH), jnp.int32),
            pltpu.VMEM((_FIRE, _CH, _D), jnp.float32),
            pltpu.SemaphoreType.DMA,
        ],
    )
    return fn(tbl, flat_idx.reshape(_ROWS // _CH, _CH))


def _mlp1_body(g_ref, c_ref, w0f_ref, w0x_ref, y_ref, st_ref):
    g = g_ref[:, :]
    gx = g[:, :16]
    gf = g[:, 16:80]
    y = (jnp.dot(gf, w0f_ref[:, :], preferred_element_type=jnp.float32)
         + jnp.dot(gx, w0x_ref[:, :], preferred_element_type=jnp.float32))
    bias = jnp.dot(c_ref[0], w0x_ref[:, :],
                   preferred_element_type=jnp.float32)
    y = (y.reshape(_QT, N_SAMPLE, 64) - bias[:, None, :]).reshape(_TILE, 64)
    y_ref[:, :] = y
    s1 = jnp.sum(y, axis=0, keepdims=True)
    s2 = jnp.sum(y * y, axis=0, keepdims=True)
    upd = jnp.concatenate([s1, s2, jnp.zeros((6, 64), jnp.float32)], axis=0)

    @pl.when(pl.program_id(0) == 0)
    def _():
        st_ref[:, :] = jnp.zeros_like(st_ref)

    st_ref[:, :] += upd


def _mlp_mid_body(y_ref, ab_ref, w_ref, o_ref, st_ref):
    a = ab_ref[0:1, :]
    b = ab_ref[1:2, :]
    h = jnp.maximum(y_ref[:, :] * a + b, 0.0)
    y = jnp.dot(h, w_ref[:, :], preferred_element_type=jnp.float32)
    o_ref[:, :] = y
    s1 = jnp.sum(y, axis=0, keepdims=True)
    s2 = jnp.sum(y * y, axis=0, keepdims=True)
    upd = jnp.concatenate([s1, s2, jnp.zeros((6, 64), jnp.float32)], axis=0)

    @pl.when(pl.program_id(0) == 0)
    def _():
        st_ref[:, :] = jnp.zeros_like(st_ref)

    st_ref[:, :] += upd


def _mlp3_body(y_ref, ab_ref, w_ref, m_ref, st_ref):
    a = ab_ref[0:1, :]
    b = ab_ref[1:2, :]
    h = jnp.maximum(y_ref[:, :] * a + b, 0.0)
    y = jnp.dot(h, w_ref[:, :], preferred_element_type=jnp.float32)
    m_ref[:, :] = jnp.max(y.reshape(_QT, N_SAMPLE, 128), axis=1)
    s1 = jnp.sum(y, axis=0, keepdims=True)
    s2 = jnp.sum(y * y, axis=0, keepdims=True)
    upd = jnp.concatenate([s1, s2, jnp.zeros((6, 128), jnp.float32)], axis=0)

    @pl.when(pl.program_id(0) == 0)
    def _():
        st_ref[:, :] = jnp.zeros_like(st_ref)

    st_ref[:, :] += upd


def _mlp4_body(m_ref, ab_ref, o_ref):
    a = ab_ref[0:1, :]
    b = ab_ref[1:2, :]
    o_ref[:, :] = jnp.maximum(m_ref[:, :] * a + b, 0.0)


def _bn_ab(st, gamma, beta):
    cnt = jnp.float32(_ROWS)
    mean = st[0] / cnt
    var = st[1] / cnt - mean * mean
    a = gamma / jnp.sqrt(var + 1e-5)
    b = beta - mean * a
    return a, b


def _pack_ab(a, b, width):
    ab = jnp.stack([a, b], axis=0)
    return jnp.concatenate([ab, jnp.zeros((6, width), jnp.float32)], axis=0)


def _run_mlp(grouped, new_xyz_pad, W0, gamma0, beta0, W1, gamma1, beta1,
             W2, gamma2, beta2):
    w0 = W0.T
    w0x = jnp.concatenate([w0[:3], jnp.zeros((13, 64), jnp.float32)], axis=0)
    w0f = w0[3:]
    y1, st1 = pl.pallas_call(
        _mlp1_body,
        grid=(_GRID,),
        in_specs=[
            pl.BlockSpec((_TILE, _D), lambda g: (g, 0)),
            pl.BlockSpec((1, _QT, 16), lambda g: (g // 4, g % 4, 0)),
            pl.BlockSpec((64, 64), lambda g: (0, 0)),
            pl.BlockSpec((16, 64), lambda g: (0, 0)),
        ],
        out_specs=(
            pl.BlockSpec((_TILE, 64), lambda g: (g, 0)),
            pl.BlockSpec((8, 64), lambda g: (0, 0)),
        ),
        out_shape=(
            jax.ShapeDtypeStruct((_ROWS, 64), jnp.float32),
            jax.ShapeDtypeStruct((8, 64), jnp.float32),
        ),
    )(grouped, new_xyz_pad, w0f, w0x)
    a1, b1 = _bn_ab(st1, gamma0, beta0)

    y2, st2 = pl.pallas_call(
        _mlp_mid_body,
        grid=(_GRID,),
        in_specs=[
            pl.BlockSpec((_TILE, 64), lambda g: (g, 0)),
            pl.BlockSpec((8, 64), lambda g: (0, 0)),
            pl.BlockSpec((64, 64), lambda g: (0, 0)),
        ],
        out_specs=(
            pl.BlockSpec((_TILE, 64), lambda g: (g, 0)),
            pl.BlockSpec((8, 64), lambda g: (0, 0)),
        ),
        out_shape=(
            jax.ShapeDtypeStruct((_ROWS, 64), jnp.float32),
            jax.ShapeDtypeStruct((8, 64), jnp.float32),
        ),
    )(y1, _pack_ab(a1, b1, 64), W1.T)
    a2, b2 = _bn_ab(st2, gamma1, beta1)

    m, st3 = pl.pallas_call(
        _mlp3_body,
        grid=(_GRID,),
        in_specs=[
            pl.BlockSpec((_TILE, 64), lambda g: (g, 0)),
            pl.BlockSpec((8, 64), lambda g: (0, 0)),
            pl.BlockSpec((64, 128), lambda g: (0, 0)),
        ],
        out_specs=(
            pl.BlockSpec((_QT, 128), lambda g: (g, 0)),
            pl.BlockSpec((8, 128), lambda g: (0, 0)),
        ),
        out_shape=(
            jax.ShapeDtypeStruct((_B * N_POINT, 128), jnp.float32),
            jax.ShapeDtypeStruct((8, 128), jnp.float32),
        ),
    )(y2, _pack_ab(a2, b2, 64), W2.T)
    a3, b3 = _bn_ab(st3, gamma2, beta2)

    out = pl.pallas_call(
        _mlp4_body,
        grid=(_GRID,),
        in_specs=[
            pl.BlockSpec((_QT, 128), lambda g: (g, 0)),
            pl.BlockSpec((8, 128), lambda g: (0, 0)),
        ],
        out_specs=pl.BlockSpec((_QT, 128), lambda g: (g, 0)),
        out_shape=jax.ShapeDtypeStruct((_B * N_POINT, 128), jnp.float32),
    )(m, _pack_ab(a3, b3, 128))
    return out.reshape(_B, N_POINT, 128)


def kernel(xyz, features, W0, gamma0, beta0, W1, gamma1, beta1, W2, gamma2,
           beta2):
    fps_idx, (cx, cy, cz), new_xyz = _run_fps(xyz)
    idx = _ball_query(xyz, new_xyz, (cx, cy, cz))

    tbl = jnp.concatenate(
        [xyz, jnp.zeros((_B, _N, 13), jnp.float32), features,
         jnp.zeros((_B, _N, _D - 80), jnp.float32)],
        axis=-1).reshape(_B * _N, _D)
    offs = (jnp.arange(_B, dtype=jnp.int32) * _N)[:, None, None]
    flat_idx = (idx + offs).reshape(_ROWS)
    grouped = _sc_gather(tbl, flat_idx)

    new_xyz_pad = jnp.concatenate(
        [new_xyz, jnp.zeros((_B, N_POINT, 13), jnp.float32)], axis=-1)
    new_features = _run_mlp(grouped, new_xyz_pad, W0, gamma0, beta0,
                            W1, gamma1, beta1, W2, gamma2, beta2)
    return (new_xyz, new_features)


# speedup vs baseline: 17.5230x; 1.1479x over previous
import functools

import jax
import jax.numpy as jnp
from jax import lax
from jax.experimental import pallas as pl
from jax.experimental.pallas import tpu as pltpu
from jax.experimental.pallas import tpu_sc as plsc

N_POINT = 1024
N_SAMPLE = 32
BALL_RADIUS = 0.2
_B = 16
_N = 4096
_D = 128
_ROWS = _B * N_POINT * N_SAMPLE
_NW = 32
_CH = 128
_FIRE = 4
_TILE = 8192
_QT = 256
_GRID = _ROWS // _TILE


def _fps_body(x_ref, y_ref, z_ref, f0_ref, idx_ref, cx_ref, cy_ref, cz_ref,
              dist_ref):
    X = x_ref[:, :]
    Y = y_ref[:, :]
    Z = z_ref[:, :]
    dist_ref[:, :] = jnp.full((_B, _N), 1e10, dtype=jnp.float32)
    iota = jax.lax.broadcasted_iota(jnp.int32, (_B, _N), 1)
    iota_s = jax.lax.broadcasted_iota(jnp.int32, (_B, N_POINT), 1)

    def body(i, far):
        oh = iota == far
        cx = jnp.sum(jnp.where(oh, X, 0.0), axis=1, keepdims=True)
        cy = jnp.sum(jnp.where(oh, Y, 0.0), axis=1, keepdims=True)
        cz = jnp.sum(jnp.where(oh, Z, 0.0), axis=1, keepdims=True)
        d = ((X - cx) ** 2 + (Y - cy) ** 2) + (Z - cz) ** 2
        nd = jnp.minimum(dist_ref[:, :], d)
        dist_ref[:, :] = nd
        m = jnp.max(nd, axis=1, keepdims=True)
        newfar = jnp.min(jnp.where(nd == m, iota, _N), axis=1,
                         keepdims=True).astype(jnp.int32)
        sel = iota_s == i
        idx_ref[:, :] = jnp.where(sel, far, idx_ref[:, :])
        cx_ref[:, :] = jnp.where(sel, cx, cx_ref[:, :])
        cy_ref[:, :] = jnp.where(sel, cy, cy_ref[:, :])
        cz_ref[:, :] = jnp.where(sel, cz, cz_ref[:, :])
        return newfar

    jax.lax.fori_loop(0, N_POINT, body, f0_ref[:, :1], unroll=False)


def _run_fps(xyz):
    X = xyz[:, :, 0]
    Y = xyz[:, :, 1]
    Z = xyz[:, :, 2]
    f0 = jax.random.randint(jax.random.key(42), (_B,), 0, _N).astype(jnp.int32)
    f0 = jnp.broadcast_to(f0[:, None], (_B, 128))
    out_shapes = (
        jax.ShapeDtypeStruct((_B, N_POINT), jnp.int32),
        jax.ShapeDtypeStruct((_B, N_POINT), jnp.float32),
        jax.ShapeDtypeStruct((_B, N_POINT), jnp.float32),
        jax.ShapeDtypeStruct((_B, N_POINT), jnp.float32),
    )
    idx, cx, cy, cz = pl.pallas_call(
        _fps_body,
        out_shape=out_shapes,
        scratch_shapes=[pltpu.VMEM((_B, _N), jnp.float32)],
    )(X, Y, Z, f0)
    new_xyz = jnp.stack([cx, cy, cz], axis=-1)
    return idx, (cx, cy, cz), new_xyz


_R2 = BALL_RADIUS * BALL_RADIUS
_QTOT = _B * N_POINT
_QPW = _QTOT // _NW
_NCHUNK = 64
_CPQ = _N // _NCHUNK


def _dist_body(q8_ref, p8_ref, md_ref):
    q8 = q8_ref[0]
    p8 = p8_ref[0]
    colmask = lax.broadcasted_iota(jnp.int32, (_QT, 8), 1) < 3
    rowmask = lax.broadcasted_iota(jnp.int32, (8, _N), 0) < 3
    qb = jnp.where(colmask, q8, 0.0).astype(jnp.bfloat16)
    pb = jnp.where(rowmask, p8, 0.0).astype(jnp.bfloat16)
    mm2 = jnp.dot(qb, pb, preferred_element_type=jnp.float32)
    qq = q8[:, 3:4]
    pp = p8[4:5, :]
    d = (qq + pp) + mm2
    d = jnp.maximum(d, 0.0)
    md_ref[:, :] = jnp.where(d > _R2, 1e10, d)


def _masked_dists(q8, p8):
    return pl.pallas_call(
        _dist_body,
        grid=(_GRID,),
        in_specs=[
            pl.BlockSpec((1, _QT, 8), lambda g: (g // 4, g % 4, 0)),
            pl.BlockSpec((1, 8, _N), lambda g: (g // 4, 0, 0)),
        ],
        out_specs=pl.BlockSpec((_QT, _N), lambda g: (g, 0)),
        out_shape=jax.ShapeDtypeStruct((_QTOT, _N), jnp.float32),
    )(q8, p8)


def _stage_a(d_at):
    lane = lax.broadcasted_iota(jnp.int32, (16,), 0)
    inf16 = jnp.full((16,), 1e10, jnp.float32)
    cbase = [(lane + 16 * jv) * _NCHUNK for jv in range(4)]

    def cmloop(s, cms):
        return tuple(
            jnp.minimum(cms[jv], plsc.load_gather(d_at, [cbase[jv] + s]))
            for jv in range(4))

    return lax.fori_loop(0, _NCHUNK, cmloop, (inf16,) * 4, unroll=8)


def _exstep(d_at, j, i0, cms, o0, o1):
        lane = lax.broadcasted_iota(jnp.int32, (16,), 0)
        inf16 = jnp.full((16,), 1e10, jnp.float32)
        best = inf16
        brow = jnp.zeros((16,), jnp.int32)
        for jv in range(4):
            upd = cms[jv] < best
            best = jnp.where(upd, cms[jv], best)
            brow = jnp.where(upd, jnp.full((16,), jv, jnp.int32), brow)
        gm = jnp.min(best)
        chunkid = brow * 16 + lane
        cstar = jnp.min(jnp.where(best == gm, chunkid,
                                  jnp.full((16,), 10 ** 6, jnp.int32)))
        big = jnp.full((16,), 10 ** 9, jnp.int32)
        posmin = big
        col = []
        base = cstar * _NCHUNK
        for k in range(4):
            pos = base + 16 * k + lane
            wk = d_at[pl.ds(base + 16 * k, 16)]
            col.append((pos, wk))
            posmin = jnp.minimum(posmin, jnp.where(wk == gm, pos, big))
        gidx_raw = jnp.min(posmin)
        ncm16 = inf16
        for k in range(4):
            pos, wk = col[k]
            ncm16 = jnp.minimum(ncm16, jnp.where(pos == gidx_raw, inf16, wk))
        plsc.store_scatter(d_at, [jnp.full((16,), gidx_raw, jnp.int32)],
                           inf16, mask=lane == 0)
        ncm = jnp.min(ncm16)
        ncms = []
        for jv in range(4):
            chunk_sel = (lane + 16 * jv) == cstar
            ncms.append(jnp.where(chunk_sel,
                                  jnp.full((16,), ncm, jnp.float32), cms[jv]))
        i0_new = jnp.where(j == 0, gidx_raw, i0)
        gidx = jnp.where(gm > _R2, i0_new, gidx_raw)
        o0 = jnp.where(lane == j, jnp.full((16,), gidx, jnp.int32), o0)
        o1 = jnp.where(lane == (j - 16), jnp.full((16,), gidx, jnp.int32), o1)
        return i0_new, tuple(ncms), o0, o1


def _select_process2(da_at, db_at, oi_ref, ql):
    cms_a = _stage_a(da_at)
    cms_b = _stage_a(db_at)
    zero16 = jnp.zeros((16,), jnp.int32)

    def exloop(j, carry):
        (i0a, ca, o0a, o1a, i0b, cb, o0b, o1b) = carry
        i0a, ca, o0a, o1a = _exstep(da_at, j, i0a, ca, o0a, o1a)
        i0b, cb, o0b, o1b = _exstep(db_at, j, i0b, cb, o0b, o1b)
        return (i0a, ca, o0a, o1a, i0b, cb, o0b, o1b)

    res = lax.fori_loop(
        0, N_SAMPLE, exloop,
        (jnp.int32(0), cms_a, zero16, zero16,
         jnp.int32(0), cms_b, zero16, zero16), unroll=False)
    oi_ref[pl.ds(ql * N_SAMPLE, 16)] = res[2]
    oi_ref[pl.ds(ql * N_SAMPLE + 16, 16)] = res[3]
    oi_ref[pl.ds((ql + 1) * N_SAMPLE, 16)] = res[6]
    oi_ref[pl.ds((ql + 1) * N_SAMPLE + 16, 16)] = res[7]


def _select_body(md_hbm, oidx_hbm, d0_v, d1_v, d2_v, d3_v, oi_v,
                 sem0, sem1, sem2, sem3):
    w = lax.axis_index("s") * 2 + lax.axis_index("c")
    q0 = w * _QPW
    pltpu.async_copy(md_hbm.at[q0], d0_v, sem0)
    pltpu.async_copy(md_hbm.at[q0 + 1], d1_v, sem1)

    def qloop(t, _):
        q = q0 + 4 * t
        pltpu.make_async_copy(md_hbm.at[q], d0_v, sem0).wait()
        pltpu.make_async_copy(md_hbm.at[q + 1], d1_v, sem1).wait()
        pltpu.async_copy(md_hbm.at[q + 2], d2_v, sem2)
        pltpu.async_copy(md_hbm.at[q + 3], d3_v, sem3)
        _select_process2(d0_v, d1_v, oi_v, 4 * t)

        @pl.when(t < _QPW // 4 - 1)
        def _():
            pltpu.async_copy(md_hbm.at[q + 4], d0_v, sem0)
            pltpu.async_copy(md_hbm.at[q + 5], d1_v, sem1)

        pltpu.make_async_copy(md_hbm.at[q + 2], d2_v, sem2).wait()
        pltpu.make_async_copy(md_hbm.at[q + 3], d3_v, sem3).wait()
        _select_process2(d2_v, d3_v, oi_v, 4 * t + 2)
        return 0

    lax.fori_loop(0, _QPW // 4, qloop, 0, unroll=False)
    pltpu.sync_copy(oi_v, oidx_hbm.at[pl.ds(q0 * N_SAMPLE,
                                            _QPW * N_SAMPLE)])


def _sc_select(md):
    mesh = plsc.VectorSubcoreMesh(core_axis_name="c", subcore_axis_name="s")
    fn = pl.kernel(
        _select_body,
        out_type=jax.ShapeDtypeStruct((_QTOT * N_SAMPLE,), jnp.int32),
        mesh=mesh,
        compiler_params=pltpu.CompilerParams(needs_layout_passes=False),
        scratch_types=[
            pltpu.VMEM((_N,), jnp.float32),
            pltpu.VMEM((_N,), jnp.float32),
            pltpu.VMEM((_N,), jnp.float32),
            pltpu.VMEM((_N,), jnp.float32),
            pltpu.VMEM((_QPW * N_SAMPLE,), jnp.int32),
            pltpu.SemaphoreType.DMA,
            pltpu.SemaphoreType.DMA,
            pltpu.SemaphoreType.DMA,
            pltpu.SemaphoreType.DMA,
        ],
    )
    return fn(md)


def _ball_query(xyz, new_xyz, cxyz):
    cx, cy, cz = cxyz
    X = xyz[:, :, 0]
    Y = xyz[:, :, 1]
    Z = xyz[:, :, 2]
    pp = X * X + Y * Y + Z * Z
    qq = cx * cx + cy * cy + cz * cz
    ones_q = jnp.ones_like(cx)
    q8 = jnp.stack([cx, cy, cz, qq, ones_q,
                    jnp.zeros_like(cx), jnp.zeros_like(cx),
                    jnp.zeros_like(cx)], axis=-1)
    p8 = jnp.stack([-2.0 * X, -2.0 * Y, -2.0 * Z, jnp.ones_like(X), pp,
                    jnp.zeros_like(X), jnp.zeros_like(X),
                    jnp.zeros_like(X)], axis=1)
    md = _masked_dists(q8, p8)
    idx = _sc_select(md)
    return idx.reshape(_B, N_POINT, N_SAMPLE)


_PER_W = _ROWS // _NW
_NSUPER = _PER_W // (_CH * _FIRE)


def _sc_gather_body(tbl_hbm, idx_hbm, out_hbm, idx_v, rows_v, gsem):
    wid = lax.axis_index("s") * 2 + lax.axis_index("c")
    nchunks = _PER_W // _CH
    pltpu.sync_copy(idx_hbm.at[pl.ds(wid * nchunks, nchunks)], idx_v)

    def super_body(s, _):
        for j in range(_FIRE):
            pltpu.async_copy(tbl_hbm.at[idx_v.at[s * _FIRE + j]],
                             rows_v.at[j], gsem)
        for j in range(_FIRE):
            pltpu.make_async_copy(tbl_hbm.at[idx_v.at[s * _FIRE + j]],
                                  rows_v.at[j], gsem).wait()
        base = wid * _PER_W + s * (_CH * _FIRE)
        for j in range(_FIRE):
            pltpu.sync_copy(rows_v.at[j],
                            out_hbm.at[pl.ds(base + j * _CH, _CH)])
        return 0

    lax.fori_loop(0, _NSUPER, super_body, 0, unroll=False)


def _sc_gather(tbl, flat_idx):
    mesh = plsc.VectorSubcoreMesh(core_axis_name="c", subcore_axis_name="s")
    fn = pl.kernel(
        _sc_gather_body,
        out_type=jax.ShapeDtypeStruct((_ROWS, _D), jnp.float32),
        mesh=mesh,
        scratch_types=[
            pltpu.VMEM((_PER_W // _CH, _CH), jnp.int32),
            pltpu.VMEM((_FIRE, _CH, _D), jnp.float32),
            pltpu.SemaphoreType.DMA,
        ],
    )
    return fn(tbl, flat_idx.reshape(_ROWS // _CH, _CH))


def _mlp1_body(g_ref, c_ref, w0f_ref, w0x_ref, y_ref, st_ref):
    g = g_ref[:, :]
    gx = g[:, :16]
    gf = g[:, 16:80]
    y = (jnp.dot(gf, w0f_ref[:, :], preferred_element_type=jnp.float32)
         + jnp.dot(gx, w0x_ref[:, :], preferred_element_type=jnp.float32))
    bias = jnp.dot(c_ref[0], w0x_ref[:, :],
                   preferred_element_type=jnp.float32)
    y = (y.reshape(_QT, N_SAMPLE, 64) - bias[:, None, :]).reshape(_TILE, 64)
    y_ref[:, :] = y
    s1 = jnp.sum(y, axis=0, keepdims=True)
    s2 = jnp.sum(y * y, axis=0, keepdims=True)
    upd = jnp.concatenate([s1, s2, jnp.zeros((6, 64), jnp.float32)], axis=0)

    @pl.when(pl.program_id(0) == 0)
    def _():
        st_ref[:, :] = jnp.zeros_like(st_ref)

    st_ref[:, :] += upd


def _mlp_mid_body(y_ref, ab_ref, w_ref, o_ref, st_ref):
    a = ab_ref[0:1, :]
    b = ab_ref[1:2, :]
    h = jnp.maximum(y_ref[:, :] * a + b, 0.0)
    y = jnp.dot(h, w_ref[:, :], preferred_element_type=jnp.float32)
    o_ref[:, :] = y
    s1 = jnp.sum(y, axis=0, keepdims=True)
    s2 = jnp.sum(y * y, axis=0, keepdims=True)
    upd = jnp.concatenate([s1, s2, jnp.zeros((6, 64), jnp.float32)], axis=0)

    @pl.when(pl.program_id(0) == 0)
    def _():
        st_ref[:, :] = jnp.zeros_like(st_ref)

    st_ref[:, :] += upd


def _mlp3_body(y_ref, ab_ref, w_ref, m_ref, st_ref):
    a = ab_ref[0:1, :]
    b = ab_ref[1:2, :]
    h = jnp.maximum(y_ref[:, :] * a + b, 0.0)
    y = jnp.dot(h, w_ref[:, :], preferred_element_type=jnp.float32)
    m_ref[:, :] = jnp.max(y.reshape(_QT, N_SAMPLE, 128), axis=1)
    s1 = jnp.sum(y, axis=0, keepdims=True)
    s2 = jnp.sum(y * y, axis=0, keepdims=True)
    upd = jnp.concatenate([s1, s2, jnp.zeros((6, 128), jnp.float32)], axis=0)

    @pl.when(pl.program_id(0) == 0)
    def _():
        st_ref[:, :] = jnp.zeros_like(st_ref)

    st_ref[:, :] += upd


def _mlp4_body(m_ref, ab_ref, o_ref):
    a = ab_ref[0:1, :]
    b = ab_ref[1:2, :]
    o_ref[:, :] = jnp.maximum(m_ref[:, :] * a + b, 0.0)


def _bn_ab(st, gamma, beta):
    cnt = jnp.float32(_ROWS)
    mean = st[0] / cnt
    var = st[1] / cnt - mean * mean
    a = gamma / jnp.sqrt(var + 1e-5)
    b = beta - mean * a
    return a, b


def _pack_ab(a, b, width):
    ab = jnp.stack([a, b], axis=0)
    return jnp.concatenate([ab, jnp.zeros((6, width), jnp.float32)], axis=0)


def _run_mlp(grouped, new_xyz_pad, W0, gamma0, beta0, W1, gamma1, beta1,
             W2, gamma2, beta2):
    w0 = W0.T
    w0x = jnp.concatenate([w0[:3], jnp.zeros((13, 64), jnp.float32)], axis=0)
    w0f = w0[3:]
    y1, st1 = pl.pallas_call(
        _mlp1_body,
        grid=(_GRID,),
        in_specs=[
            pl.BlockSpec((_TILE, _D), lambda g: (g, 0)),
            pl.BlockSpec((1, _QT, 16), lambda g: (g // 4, g % 4, 0)),
            pl.BlockSpec((64, 64), lambda g: (0, 0)),
            pl.BlockSpec((16, 64), lambda g: (0, 0)),
        ],
        out_specs=(
            pl.BlockSpec((_TILE, 64), lambda g: (g, 0)),
            pl.BlockSpec((8, 64), lambda g: (0, 0)),
        ),
        out_shape=(
            jax.ShapeDtypeStruct((_ROWS, 64), jnp.float32),
            jax.ShapeDtypeStruct((8, 64), jnp.float32),
        ),
    )(grouped, new_xyz_pad, w0f, w0x)
    a1, b1 = _bn_ab(st1, gamma0, beta0)

    y2, st2 = pl.pallas_call(
        _mlp_mid_body,
        grid=(_GRID,),
        in_specs=[
            pl.BlockSpec((_TILE, 64), lambda g: (g, 0)),
            pl.BlockSpec((8, 64), lambda g: (0, 0)),
            pl.BlockSpec((64, 64), lambda g: (0, 0)),
        ],
        out_specs=(
            pl.BlockSpec((_TILE, 64), lambda g: (g, 0)),
            pl.BlockSpec((8, 64), lambda g: (0, 0)),
        ),
        out_shape=(
            jax.ShapeDtypeStruct((_ROWS, 64), jnp.float32),
            jax.ShapeDtypeStruct((8, 64), jnp.float32),
        ),
    )(y1, _pack_ab(a1, b1, 64), W1.T)
    a2, b2 = _bn_ab(st2, gamma1, beta1)

    m, st3 = pl.pallas_call(
        _mlp3_body,
        grid=(_GRID,),
        in_specs=[
            pl.BlockSpec((_TILE, 64), lambda g: (g, 0)),
            pl.BlockSpec((8, 64), lambda g: (0, 0)),
            pl.BlockSpec((64, 128), lambda g: (0, 0)),
        ],
        out_specs=(
            pl.BlockSpec((_QT, 128), lambda g: (g, 0)),
            pl.BlockSpec((8, 128), lambda g: (0, 0)),
        ),
        out_shape=(
            jax.ShapeDtypeStruct((_B * N_POINT, 128), jnp.float32),
            jax.ShapeDtypeStruct((8, 128), jnp.float32),
        ),
    )(y2, _pack_ab(a2, b2, 64), W2.T)
    a3, b3 = _bn_ab(st3, gamma2, beta2)

    out = pl.pallas_call(
        _mlp4_body,
        grid=(_GRID,),
        in_specs=[
            pl.BlockSpec((_QT, 128), lambda g: (g, 0)),
            pl.BlockSpec((8, 128), lambda g: (0, 0)),
        ],
        out_specs=pl.BlockSpec((_QT, 128), lambda g: (g, 0)),
        out_shape=jax.ShapeDtypeStruct((_B * N_POINT, 128), jnp.float32),
    )(m, _pack_ab(a3, b3, 128))
    return out.reshape(_B, N_POINT, 128)


def kernel(xyz, features, W0, gamma0, beta0, W1, gamma1, beta1, W2, gamma2,
           beta2):
    fps_idx, (cx, cy, cz), new_xyz = _run_fps(xyz)
    idx = _ball_query(xyz, new_xyz, (cx, cy, cz))

    tbl = jnp.concatenate(
        [xyz, jnp.zeros((_B, _N, 13), jnp.float32), features,
         jnp.zeros((_B, _N, _D - 80), jnp.float32)],
        axis=-1).reshape(_B * _N, _D)
    offs = (jnp.arange(_B, dtype=jnp.int32) * _N)[:, None, None]
    flat_idx = (idx + offs).reshape(_ROWS)
    grouped = _sc_gather(tbl, flat_idx)

    new_xyz_pad = jnp.concatenate(
        [new_xyz, jnp.zeros((_B, N_POINT, 13), jnp.float32)], axis=-1)
    new_features = _run_mlp(grouped, new_xyz_pad, W0, gamma0, beta0,
                            W1, gamma1, beta1, W2, gamma2, beta2)
    return (new_xyz, new_features)


# speedup vs baseline: 17.7337x; 1.0120x over previous
import functools

import jax
import jax.numpy as jnp
from jax import lax
from jax.experimental import pallas as pl
from jax.experimental.pallas import tpu as pltpu
from jax.experimental.pallas import tpu_sc as plsc

N_POINT = 1024
N_SAMPLE = 32
BALL_RADIUS = 0.2
_B = 16
_N = 4096
_D = 128
_ROWS = _B * N_POINT * N_SAMPLE
_NW = 32
_CH = 128
_FIRE = 4
_TILE = 8192
_QT = 256
_GRID = _ROWS // _TILE


def _fps_body(x_ref, y_ref, z_ref, f0_ref, idx_ref, cx_ref, cy_ref, cz_ref,
              dist_ref):
    X = x_ref[:, :]
    Y = y_ref[:, :]
    Z = z_ref[:, :]
    dist_ref[:, :] = jnp.full((_B, _N), 1e10, dtype=jnp.float32)
    iota = jax.lax.broadcasted_iota(jnp.int32, (_B, _N), 1)
    iota_s = jax.lax.broadcasted_iota(jnp.int32, (_B, N_POINT), 1)

    def body(i, far):
        oh = iota == far
        cx = jnp.sum(jnp.where(oh, X, 0.0), axis=1, keepdims=True)
        cy = jnp.sum(jnp.where(oh, Y, 0.0), axis=1, keepdims=True)
        cz = jnp.sum(jnp.where(oh, Z, 0.0), axis=1, keepdims=True)
        d = ((X - cx) ** 2 + (Y - cy) ** 2) + (Z - cz) ** 2
        nd = jnp.minimum(dist_ref[:, :], d)
        dist_ref[:, :] = nd
        m = jnp.max(nd, axis=1, keepdims=True)
        newfar = jnp.min(jnp.where(nd == m, iota, _N), axis=1,
                         keepdims=True).astype(jnp.int32)
        sel = iota_s == i
        idx_ref[:, :] = jnp.where(sel, far, idx_ref[:, :])
        cx_ref[:, :] = jnp.where(sel, cx, cx_ref[:, :])
        cy_ref[:, :] = jnp.where(sel, cy, cy_ref[:, :])
        cz_ref[:, :] = jnp.where(sel, cz, cz_ref[:, :])
        return newfar

    jax.lax.fori_loop(0, N_POINT, body, f0_ref[:, :1], unroll=False)


def _run_fps(xyz):
    X = xyz[:, :, 0]
    Y = xyz[:, :, 1]
    Z = xyz[:, :, 2]
    f0 = jax.random.randint(jax.random.key(42), (_B,), 0, _N).astype(jnp.int32)
    f0 = jnp.broadcast_to(f0[:, None], (_B, 128))
    out_shapes = (
        jax.ShapeDtypeStruct((_B, N_POINT), jnp.int32),
        jax.ShapeDtypeStruct((_B, N_POINT), jnp.float32),
        jax.ShapeDtypeStruct((_B, N_POINT), jnp.float32),
        jax.ShapeDtypeStruct((_B, N_POINT), jnp.float32),
    )
    idx, cx, cy, cz = pl.pallas_call(
        _fps_body,
        out_shape=out_shapes,
        scratch_shapes=[pltpu.VMEM((_B, _N), jnp.float32)],
    )(X, Y, Z, f0)
    new_xyz = jnp.stack([cx, cy, cz], axis=-1)
    return idx, (cx, cy, cz), new_xyz


_R2 = BALL_RADIUS * BALL_RADIUS
_QTOT = _B * N_POINT
_QPW = _QTOT // _NW
_NCHUNK = 64
_CPQ = _N // _NCHUNK


def _dist_body(q8_ref, p8_ref, md_ref):
    q8 = q8_ref[0]
    p8 = p8_ref[0]
    colmask = lax.broadcasted_iota(jnp.int32, (_QT, 8), 1) < 3
    rowmask = lax.broadcasted_iota(jnp.int32, (8, _N), 0) < 3
    qb = jnp.where(colmask, q8, 0.0).astype(jnp.bfloat16)
    pb = jnp.where(rowmask, p8, 0.0).astype(jnp.bfloat16)
    mm2 = jnp.dot(qb, pb, preferred_element_type=jnp.float32)
    qq = q8[:, 3:4]
    pp = p8[4:5, :]
    d = (qq + pp) + mm2
    d = jnp.maximum(d, 0.0)
    md_ref[:, :] = jnp.where(d > _R2, 1e10, d)


def _masked_dists(q8, p8):
    nb = q8.shape[0]
    return pl.pallas_call(
        _dist_body,
        grid=(nb * 4,),
        in_specs=[
            pl.BlockSpec((1, _QT, 8), lambda g: (g // 4, g % 4, 0)),
            pl.BlockSpec((1, 8, _N), lambda g: (g // 4, 0, 0)),
        ],
        out_specs=pl.BlockSpec((_QT, _N), lambda g: (g, 0)),
        out_shape=jax.ShapeDtypeStruct((nb * N_POINT, _N), jnp.float32),
    )(q8, p8)


def _stage_a(d_at):
    lane = lax.broadcasted_iota(jnp.int32, (16,), 0)
    inf16 = jnp.full((16,), 1e10, jnp.float32)
    cbase = [(lane + 16 * jv) * _NCHUNK for jv in range(4)]

    def cmloop(s, cms):
        return tuple(
            jnp.minimum(cms[jv], plsc.load_gather(d_at, [cbase[jv] + s]))
            for jv in range(4))

    return lax.fori_loop(0, _NCHUNK, cmloop, (inf16,) * 4, unroll=8)


def _exstep(d_at, j, i0, cms, o0, o1):
        lane = lax.broadcasted_iota(jnp.int32, (16,), 0)
        inf16 = jnp.full((16,), 1e10, jnp.float32)
        best = inf16
        brow = jnp.zeros((16,), jnp.int32)
        for jv in range(4):
            upd = cms[jv] < best
            best = jnp.where(upd, cms[jv], best)
            brow = jnp.where(upd, jnp.full((16,), jv, jnp.int32), brow)
        gm = jnp.min(best)
        chunkid = brow * 16 + lane
        cstar = jnp.min(jnp.where(best == gm, chunkid,
                                  jnp.full((16,), 10 ** 6, jnp.int32)))
        big = jnp.full((16,), 10 ** 9, jnp.int32)
        posmin = big
        col = []
        base = cstar * _NCHUNK
        for k in range(4):
            pos = base + 16 * k + lane
            wk = d_at[pl.ds(base + 16 * k, 16)]
            col.append((pos, wk))
            posmin = jnp.minimum(posmin, jnp.where(wk == gm, pos, big))
        gidx_raw = jnp.min(posmin)
        ncm16 = inf16
        for k in range(4):
            pos, wk = col[k]
            ncm16 = jnp.minimum(ncm16, jnp.where(pos == gidx_raw, inf16, wk))
        plsc.store_scatter(d_at, [jnp.full((16,), gidx_raw, jnp.int32)],
                           inf16, mask=lane == 0)
        ncm = jnp.min(ncm16)
        ncms = []
        for jv in range(4):
            chunk_sel = (lane + 16 * jv) == cstar
            ncms.append(jnp.where(chunk_sel,
                                  jnp.full((16,), ncm, jnp.float32), cms[jv]))
        i0_new = jnp.where(j == 0, gidx_raw, i0)
        gidx = jnp.where(gm > _R2, i0_new, gidx_raw)
        o0 = jnp.where(lane == j, jnp.full((16,), gidx, jnp.int32), o0)
        o1 = jnp.where(lane == (j - 16), jnp.full((16,), gidx, jnp.int32), o1)
        return i0_new, tuple(ncms), o0, o1


def _select_process2(da_at, db_at, oi_ref, ql):
    cms_a = _stage_a(da_at)
    cms_b = _stage_a(db_at)
    zero16 = jnp.zeros((16,), jnp.int32)

    def exloop(j, carry):
        (i0a, ca, o0a, o1a, i0b, cb, o0b, o1b) = carry
        i0a, ca, o0a, o1a = _exstep(da_at, j, i0a, ca, o0a, o1a)
        i0b, cb, o0b, o1b = _exstep(db_at, j, i0b, cb, o0b, o1b)
        return (i0a, ca, o0a, o1a, i0b, cb, o0b, o1b)

    res = lax.fori_loop(
        0, N_SAMPLE, exloop,
        (jnp.int32(0), cms_a, zero16, zero16,
         jnp.int32(0), cms_b, zero16, zero16), unroll=False)
    oi_ref[pl.ds(ql * N_SAMPLE, 16)] = res[2]
    oi_ref[pl.ds(ql * N_SAMPLE + 16, 16)] = res[3]
    oi_ref[pl.ds((ql + 1) * N_SAMPLE, 16)] = res[6]
    oi_ref[pl.ds((ql + 1) * N_SAMPLE + 16, 16)] = res[7]


def _make_select_body(qpw):
    def _select_body(md_hbm, oidx_hbm, d0_v, d1_v, d2_v, d3_v, oi_v,
                     sem0, sem1, sem2, sem3):
        w = lax.axis_index("s") * 2 + lax.axis_index("c")
        q0 = w * qpw
        pltpu.async_copy(md_hbm.at[q0], d0_v, sem0)
        pltpu.async_copy(md_hbm.at[q0 + 1], d1_v, sem1)

        def qloop(t, _):
            q = q0 + 4 * t
            pltpu.make_async_copy(md_hbm.at[q], d0_v, sem0).wait()
            pltpu.make_async_copy(md_hbm.at[q + 1], d1_v, sem1).wait()
            pltpu.async_copy(md_hbm.at[q + 2], d2_v, sem2)
            pltpu.async_copy(md_hbm.at[q + 3], d3_v, sem3)
            _select_process2(d0_v, d1_v, oi_v, 4 * t)

            @pl.when(t < qpw // 4 - 1)
            def _():
                pltpu.async_copy(md_hbm.at[q + 4], d0_v, sem0)
                pltpu.async_copy(md_hbm.at[q + 5], d1_v, sem1)

            pltpu.make_async_copy(md_hbm.at[q + 2], d2_v, sem2).wait()
            pltpu.make_async_copy(md_hbm.at[q + 3], d3_v, sem3).wait()
            _select_process2(d2_v, d3_v, oi_v, 4 * t + 2)
            return 0

        lax.fori_loop(0, qpw // 4, qloop, 0, unroll=False)
        pltpu.sync_copy(oi_v, oidx_hbm.at[pl.ds(q0 * N_SAMPLE,
                                                qpw * N_SAMPLE)])

    return _select_body


def _sc_select(md):
    nq = md.shape[0]
    qpw = nq // _NW
    mesh = plsc.VectorSubcoreMesh(core_axis_name="c", subcore_axis_name="s")
    fn = pl.kernel(
        _make_select_body(qpw),
        out_type=jax.ShapeDtypeStruct((nq * N_SAMPLE,), jnp.int32),
        mesh=mesh,
        compiler_params=pltpu.CompilerParams(needs_layout_passes=False),
        scratch_types=[
            pltpu.VMEM((_N,), jnp.float32),
            pltpu.VMEM((_N,), jnp.float32),
            pltpu.VMEM((_N,), jnp.float32),
            pltpu.VMEM((_N,), jnp.float32),
            pltpu.VMEM((qpw * N_SAMPLE,), jnp.int32),
            pltpu.SemaphoreType.DMA,
            pltpu.SemaphoreType.DMA,
            pltpu.SemaphoreType.DMA,
            pltpu.SemaphoreType.DMA,
        ],
    )
    return fn(md)


def _ball_query(xyz, new_xyz, cxyz):
    cx, cy, cz = cxyz
    X = xyz[:, :, 0]
    Y = xyz[:, :, 1]
    Z = xyz[:, :, 2]
    pp = X * X + Y * Y + Z * Z
    qq = cx * cx + cy * cy + cz * cz
    ones_q = jnp.ones_like(cx)
    q8 = jnp.stack([cx, cy, cz, qq, ones_q,
                    jnp.zeros_like(cx), jnp.zeros_like(cx),
                    jnp.zeros_like(cx)], axis=-1)
    p8 = jnp.stack([-2.0 * X, -2.0 * Y, -2.0 * Z, jnp.ones_like(X), pp,
                    jnp.zeros_like(X), jnp.zeros_like(X),
                    jnp.zeros_like(X)], axis=1)
    parts = []
    for qtr in range(4):
        sl = slice(qtr * 4, qtr * 4 + 4)
        md = _masked_dists(q8[sl], p8[sl])
        parts.append(_sc_select(md))
    idx = jnp.concatenate(parts)
    return idx.reshape(_B, N_POINT, N_SAMPLE)


_PER_W = _ROWS // _NW
_NSUPER = _PER_W // (_CH * _FIRE)


def _sc_gather_body(tbl_hbm, idx_hbm, out_hbm, idx_v, rows_v, gsem):
    wid = lax.axis_index("s") * 2 + lax.axis_index("c")
    nchunks = _PER_W // _CH
    pltpu.sync_copy(idx_hbm.at[pl.ds(wid * nchunks, nchunks)], idx_v)

    def super_body(s, _):
        for j in range(_FIRE):
            pltpu.async_copy(tbl_hbm.at[idx_v.at[s * _FIRE + j]],
                             rows_v.at[j], gsem)
        for j in range(_FIRE):
            pltpu.make_async_copy(tbl_hbm.at[idx_v.at[s * _FIRE + j]],
                                  rows_v.at[j], gsem).wait()
        base = wid * _PER_W + s * (_CH * _FIRE)
        for j in range(_FIRE):
            pltpu.sync_copy(rows_v.at[j],
                            out_hbm.at[pl.ds(base + j * _CH, _CH)])
        return 0

    lax.fori_loop(0, _NSUPER, super_body, 0, unroll=False)


def _sc_gather(tbl, flat_idx):
    mesh = plsc.VectorSubcoreMesh(core_axis_name="c", subcore_axis_name="s")
    fn = pl.kernel(
        _sc_gather_body,
        out_type=jax.ShapeDtypeStruct((_ROWS, _D), jnp.float32),
        mesh=mesh,
        scratch_types=[
            pltpu.VMEM((_PER_W // _CH, _CH), jnp.int32),
            pltpu.VMEM((_FIRE, _CH, _D), jnp.float32),
            pltpu.SemaphoreType.DMA,
        ],
    )
    return fn(tbl, flat_idx.reshape(_ROWS // _CH, _CH))


def _mlp1_body(g_ref, c_ref, w0f_ref, w0x_ref, y_ref, st_ref):
    g = g_ref[:, :]
    gx = g[:, :16]
    gf = g[:, 16:80]
    y = (jnp.dot(gf, w0f_ref[:, :], preferred_element_type=jnp.float32)
         + jnp.dot(gx, w0x_ref[:, :], preferred_element_type=jnp.float32))
    bias = jnp.dot(c_ref[0], w0x_ref[:, :],
                   preferred_element_type=jnp.float32)
    y = (y.reshape(_QT, N_SAMPLE, 64) - bias[:, None, :]).reshape(_TILE, 64)
    y_ref[:, :] = y
    s1 = jnp.sum(y, axis=0, keepdims=True)
    s2 = jnp.sum(y * y, axis=0, keepdims=True)
    upd = jnp.concatenate([s1, s2, jnp.zeros((6, 64), jnp.float32)], axis=0)

    @pl.when(pl.program_id(0) == 0)
    def _():
        st_ref[:, :] = jnp.zeros_like(st_ref)

    st_ref[:, :] += upd


def _mlp_mid_body(y_ref, ab_ref, w_ref, o_ref, st_ref):
    a = ab_ref[0:1, :]
    b = ab_ref[1:2, :]
    h = jnp.maximum(y_ref[:, :] * a + b, 0.0)
    y = jnp.dot(h, w_ref[:, :], preferred_element_type=jnp.float32)
    o_ref[:, :] = y
    s1 = jnp.sum(y, axis=0, keepdims=True)
    s2 = jnp.sum(y * y, axis=0, keepdims=True)
    upd = jnp.concatenate([s1, s2, jnp.zeros((6, 64), jnp.float32)], axis=0)

    @pl.when(pl.program_id(0) == 0)
    def _():
        st_ref[:, :] = jnp.zeros_like(st_ref)

    st_ref[:, :] += upd


def _mlp3_body(y_ref, ab_ref, w_ref, m_ref, st_ref):
    a = ab_ref[0:1, :]
    b = ab_ref[1:2, :]
    h = jnp.maximum(y_ref[:, :] * a + b, 0.0)
    y = jnp.dot(h, w_ref[:, :], preferred_element_type=jnp.float32)
    m_ref[:, :] = jnp.max(y.reshape(_QT, N_SAMPLE, 128), axis=1)
    s1 = jnp.sum(y, axis=0, keepdims=True)
    s2 = jnp.sum(y * y, axis=0, keepdims=True)
    upd = jnp.concatenate([s1, s2, jnp.zeros((6, 128), jnp.float32)], axis=0)

    @pl.when(pl.program_id(0) == 0)
    def _():
        st_ref[:, :] = jnp.zeros_like(st_ref)

    st_ref[:, :] += upd


def _mlp4_body(m_ref, ab_ref, o_ref):
    a = ab_ref[0:1, :]
    b = ab_ref[1:2, :]
    o_ref[:, :] = jnp.maximum(m_ref[:, :] * a + b, 0.0)


def _bn_ab(st, gamma, beta):
    cnt = jnp.float32(_ROWS)
    mean = st[0] / cnt
    var = st[1] / cnt - mean * mean
    a = gamma / jnp.sqrt(var + 1e-5)
    b = beta - mean * a
    return a, b


def _pack_ab(a, b, width):
    ab = jnp.stack([a, b], axis=0)
    return jnp.concatenate([ab, jnp.zeros((6, width), jnp.float32)], axis=0)


def _run_mlp(grouped, new_xyz_pad, W0, gamma0, beta0, W1, gamma1, beta1,
             W2, gamma2, beta2):
    w0 = W0.T
    w0x = jnp.concatenate([w0[:3], jnp.zeros((13, 64), jnp.float32)], axis=0)
    w0f = w0[3:]
    y1, st1 = pl.pallas_call(
        _mlp1_body,
        grid=(_GRID,),
        in_specs=[
            pl.BlockSpec((_TILE, _D), lambda g: (g, 0)),
            pl.BlockSpec((1, _QT, 16), lambda g: (g // 4, g % 4, 0)),
            pl.BlockSpec((64, 64), lambda g: (0, 0)),
            pl.BlockSpec((16, 64), lambda g: (0, 0)),
        ],
        out_specs=(
            pl.BlockSpec((_TILE, 64), lambda g: (g, 0)),
            pl.BlockSpec((8, 64), lambda g: (0, 0)),
        ),
        out_shape=(
            jax.ShapeDtypeStruct((_ROWS, 64), jnp.float32),
            jax.ShapeDtypeStruct((8, 64), jnp.float32),
        ),
    )(grouped, new_xyz_pad, w0f, w0x)
    a1, b1 = _bn_ab(st1, gamma0, beta0)

    y2, st2 = pl.pallas_call(
        _mlp_mid_body,
        grid=(_GRID,),
        in_specs=[
            pl.BlockSpec((_TILE, 64), lambda g: (g, 0)),
            pl.BlockSpec((8, 64), lambda g: (0, 0)),
            pl.BlockSpec((64, 64), lambda g: (0, 0)),
        ],
        out_specs=(
            pl.BlockSpec((_TILE, 64), lambda g: (g, 0)),
            pl.BlockSpec((8, 64), lambda g: (0, 0)),
        ),
        out_shape=(
            jax.ShapeDtypeStruct((_ROWS, 64), jnp.float32),
            jax.ShapeDtypeStruct((8, 64), jnp.float32),
        ),
    )(y1, _pack_ab(a1, b1, 64), W1.T)
    a2, b2 = _bn_ab(st2, gamma1, beta1)

    m, st3 = pl.pallas_call(
        _mlp3_body,
        grid=(_GRID,),
        in_specs=[
            pl.BlockSpec((_TILE, 64), lambda g: (g, 0)),
            pl.BlockSpec((8, 64), lambda g: (0, 0)),
            pl.BlockSpec((64, 128), lambda g: (0, 0)),
        ],
        out_specs=(
            pl.BlockSpec((_QT, 128), lambda g: (g, 0)),
            pl.BlockSpec((8, 128), lambda g: (0, 0)),
        ),
        out_shape=(
            jax.ShapeDtypeStruct((_B * N_POINT, 128), jnp.float32),
            jax.ShapeDtypeStruct((8, 128), jnp.float32),
        ),
    )(y2, _pack_ab(a2, b2, 64), W2.T)
    a3, b3 = _bn_ab(st3, gamma2, beta2)

    out = pl.pallas_call(
        _mlp4_body,
        grid=(_GRID,),
        in_specs=[
            pl.BlockSpec((_QT, 128), lambda g: (g, 0)),
            pl.BlockSpec((8, 128), lambda g: (0, 0)),
        ],
        out_specs=pl.BlockSpec((_QT, 128), lambda g: (g, 0)),
        out_shape=jax.ShapeDtypeStruct((_B * N_POINT, 128), jnp.float32),
    )(m, _pack_ab(a3, b3, 128))
    return out.reshape(_B, N_POINT, 128)


def kernel(xyz, features, W0, gamma0, beta0, W1, gamma1, beta1, W2, gamma2,
           beta2):
    fps_idx, (cx, cy, cz), new_xyz = _run_fps(xyz)
    idx = _ball_query(xyz, new_xyz, (cx, cy, cz))

    tbl = jnp.concatenate(
        [xyz, jnp.zeros((_B, _N, 13), jnp.float32), features,
         jnp.zeros((_B, _N, _D - 80), jnp.float32)],
        axis=-1).reshape(_B * _N, _D)
    offs = (jnp.arange(_B, dtype=jnp.int32) * _N)[:, None, None]
    flat_idx = (idx + offs).reshape(_ROWS)
    grouped = _sc_gather(tbl, flat_idx)

    new_xyz_pad = jnp.concatenate(
        [new_xyz, jnp.zeros((_B, N_POINT, 13), jnp.float32)], axis=-1)
    new_features = _run_mlp(grouped, new_xyz_pad, W0, gamma0, beta0,
                            W1, gamma1, beta1, W2, gamma2, beta2)
    return (new_xyz, new_features)
